# Initial kernel scaffold; baseline (speedup 1.0000x reference)
import jax
import jax.numpy as jnp
from jax.experimental import pallas as pl


def kernel(coors, points, features, f_cluster, W_r0, g_r0, b_r0, W_r1, g_r1, b_r1, W_v0, g_v0, b_v0, W_v1, g_v1, b_v1, W_f, b_f):
    raise NotImplementedError("write your pallas kernel here")


# speedup vs baseline: 1.3117x; 1.3117x over previous
import functools
import jax
import jax.numpy as jnp
from jax import lax
from jax.experimental import pallas as pl
from jax.experimental.pallas import tpu as pltpu
from jax.experimental.pallas import tpu_sc as plsc

N = 320000
K = 10000
KP = 10240
F32 = jnp.float32

_BI = 3200
_NBI = N // _BI


def _inv_body(coors_ref, inv_ref, carry):
    i = pl.program_id(0)
    blk = coors_ref[0]

    @pl.when(i == 0)
    def _():
        carry[0] = blk[0, 0]
        carry[1] = 0

    prev = carry[0]
    shifted = jnp.roll(blk, 1, axis=1)
    col = lax.broadcasted_iota(jnp.int32, blk.shape, 1)
    shifted = jnp.where(col == 0, prev, shifted)
    flags = (blk != shifted).astype(jnp.int32)
    x = flags
    sh = 1
    while sh < _BI:
        x = x + jnp.where(col >= sh, jnp.roll(x, sh, axis=1), 0)
        sh *= 2
    inv_ref[0] = carry[1] + x
    carry[1] = carry[1] + jnp.sum(flags)
    carry[0] = blk[0, _BI - 1]


def _compute_inv(coors_i32):
    c3 = coors_i32.reshape(_NBI, 1, _BI)
    out = pl.pallas_call(
        _inv_body,
        grid=(_NBI,),
        in_specs=[pl.BlockSpec((1, 1, _BI), lambda i: (i, 0, 0))],
        out_specs=pl.BlockSpec((1, 1, _BI), lambda i: (i, 0, 0)),
        out_shape=jax.ShapeDtypeStruct((_NBI, 1, _BI), jnp.int32),
        scratch_shapes=[pltpu.SMEM((2,), jnp.int32)],
    )(c3)
    return out.reshape(N)


def _bn_affine(S, M, W, g, b):
    mu = jnp.dot(S, W, preferred_element_type=F32, precision=lax.Precision.HIGHEST) / N
    ex2 = jnp.sum(W * jnp.dot(M, W, preferred_element_type=F32, precision=lax.Precision.HIGHEST), axis=0, keepdims=True) / N
    var = ex2 - mu * mu
    a = g / jnp.sqrt(var + 1e-5)
    c = b - mu * a
    return a, c


_BA = 2000
_NBA = N // _BA


def _kA_body(fc_ref, S1_ref, M1_ref):
    i = pl.program_id(0)

    @pl.when(i == 0)
    def _():
        S1_ref[...] = jnp.zeros_like(S1_ref)
        M1_ref[...] = jnp.zeros_like(M1_ref)

    fc = fc_ref[...]
    S1_ref[...] += jnp.sum(fc, axis=0, keepdims=True)
    M1_ref[...] += lax.dot_general(fc, fc, (((0,), (0,)), ((), ())),
                                   preferred_element_type=F32, precision=lax.Precision.HIGHEST)


def _run_kA(fc):
    return pl.pallas_call(
        _kA_body,
        grid=(_NBA,),
        in_specs=[pl.BlockSpec((_BA, 3), lambda i: (i, 0))],
        out_specs=[pl.BlockSpec((1, 3), lambda i: (0, 0)),
                   pl.BlockSpec((3, 3), lambda i: (0, 0))],
        out_shape=[jax.ShapeDtypeStruct((1, 3), F32),
                   jax.ShapeDtypeStruct((3, 3), F32)],
    )(fc)


def _kB_body(fc_ref, S1_ref, M1_ref, Wr0_ref, g0_ref, b0_ref, S2_ref, M2_ref):
    i = pl.program_id(0)

    @pl.when(i == 0)
    def _():
        S2_ref[...] = jnp.zeros_like(S2_ref)
        M2_ref[...] = jnp.zeros_like(M2_ref)

    Wr0 = Wr0_ref[...]
    a0, c0 = _bn_affine(S1_ref[...], M1_ref[...], Wr0, g0_ref[...], b0_ref[...])
    fc = fc_ref[...]
    h = jnp.maximum(jnp.dot(fc, Wr0, preferred_element_type=F32) * a0 + c0, 0.0)
    S2_ref[...] += jnp.sum(h, axis=0, keepdims=True)
    M2_ref[...] += lax.dot_general(h, h, (((0,), (0,)), ((), ())),
                                   preferred_element_type=F32, precision=lax.Precision.HIGHEST)


def _run_kB(fc, S1, M1, Wr0, g0, b0):
    return pl.pallas_call(
        _kB_body,
        grid=(_NBA,),
        in_specs=[pl.BlockSpec((_BA, 3), lambda i: (i, 0)),
                  pl.BlockSpec((1, 3), lambda i: (0, 0)),
                  pl.BlockSpec((3, 3), lambda i: (0, 0)),
                  pl.BlockSpec((3, 16), lambda i: (0, 0)),
                  pl.BlockSpec((1, 16), lambda i: (0, 0)),
                  pl.BlockSpec((1, 16), lambda i: (0, 0))],
        out_specs=[pl.BlockSpec((1, 16), lambda i: (0, 0)),
                   pl.BlockSpec((16, 16), lambda i: (0, 0))],
        out_shape=[jax.ShapeDtypeStruct((1, 16), F32),
                   jax.ShapeDtypeStruct((16, 16), F32)],
    )(fc, S1, M1, Wr0, g0, b0)


def _kC_body(fc_ref, f0_ref, S1_ref, M1_ref, Wr0_ref, g0_ref, b0_ref,
             S2_ref, M2_ref, Wr1_ref, g1_ref, b1_ref, S3_ref, M3_ref):
    i = pl.program_id(0)

    @pl.when(i == 0)
    def _():
        S3_ref[...] = jnp.zeros_like(S3_ref)
        M3_ref[...] = jnp.zeros_like(M3_ref)

    Wr0 = Wr0_ref[...]
    Wr1 = Wr1_ref[...]
    a0, c0 = _bn_affine(S1_ref[...], M1_ref[...], Wr0, g0_ref[...], b0_ref[...])
    a1, c1 = _bn_affine(S2_ref[...], M2_ref[...], Wr1, g1_ref[...], b1_ref[...])
    fc = fc_ref[...]
    h = jnp.maximum(jnp.dot(fc, Wr0, preferred_element_type=F32) * a0 + c0, 0.0)
    rel = jnp.maximum(jnp.dot(h, Wr1, preferred_element_type=F32) * a1 + c1, 0.0)
    feats = f0_ref[...] * rel
    S3_ref[...] += jnp.sum(feats, axis=0, keepdims=True)
    M3_ref[...] += lax.dot_general(feats, feats, (((0,), (0,)), ((), ())),
                                   preferred_element_type=F32, precision=lax.Precision.HIGHEST)


def _run_kC(fc, f0, S1, M1, Wr0, g0, b0, S2, M2, Wr1, g1, b1):
    cmap = lambda i: (0, 0)
    return pl.pallas_call(
        _kC_body,
        grid=(_NBA,),
        in_specs=[pl.BlockSpec((_BA, 3), lambda i: (i, 0)),
                  pl.BlockSpec((_BA, 64), lambda i: (i, 0)),
                  pl.BlockSpec((1, 3), cmap), pl.BlockSpec((3, 3), cmap),
                  pl.BlockSpec((3, 16), cmap), pl.BlockSpec((1, 16), cmap),
                  pl.BlockSpec((1, 16), cmap), pl.BlockSpec((1, 16), cmap),
                  pl.BlockSpec((16, 16), cmap), pl.BlockSpec((16, 64), cmap),
                  pl.BlockSpec((1, 64), cmap), pl.BlockSpec((1, 64), cmap)],
        out_specs=[pl.BlockSpec((1, 64), cmap),
                   pl.BlockSpec((64, 64), cmap)],
        out_shape=[jax.ShapeDtypeStruct((1, 64), F32),
                   jax.ShapeDtypeStruct((64, 64), F32)],
    )(fc, f0, S1, M1, Wr0, g0, b0, S2, M2, Wr1, g1, b1)


_BD = 1000
_NBD = N // _BD


def _kD_body(fc_ref, f0_ref, S1_ref, M1_ref, Wr0_ref, g0_ref, b0_ref,
             S2_ref, M2_ref, Wr1_ref, g1_ref, b1_ref,
             S3_ref, M3_ref, Wv0_ref, g2_ref, b2_ref,
             pf0_ref, Spf_ref, Mpf_ref, aff_scr):
    i = pl.program_id(0)

    @pl.when(i == 0)
    def _():
        Spf_ref[...] = jnp.zeros_like(Spf_ref)
        Mpf_ref[...] = jnp.zeros_like(Mpf_ref)
        a2, c2 = _bn_affine(S3_ref[...], M3_ref[...], Wv0_ref[...],
                            g2_ref[...], b2_ref[...])
        aff_scr[0:1, :] = a2
        aff_scr[1:2, :] = c2

    Wr0 = Wr0_ref[...]
    Wr1 = Wr1_ref[...]
    a0, c0 = _bn_affine(S1_ref[...], M1_ref[...], Wr0, g0_ref[...], b0_ref[...])
    a1, c1 = _bn_affine(S2_ref[...], M2_ref[...], Wr1, g1_ref[...], b1_ref[...])
    fc = fc_ref[...]
    h = jnp.maximum(jnp.dot(fc, Wr0, preferred_element_type=F32) * a0 + c0, 0.0)
    rel = jnp.maximum(jnp.dot(h, Wr1, preferred_element_type=F32) * a1 + c1, 0.0)
    feats = f0_ref[...] * rel
    t2 = jnp.dot(feats, Wv0_ref[...], preferred_element_type=F32)
    pf0 = jnp.maximum(t2 * aff_scr[0:1, :] + aff_scr[1:2, :], 0.0)
    pf0_ref[...] = pf0
    Spf_ref[...] += jnp.sum(pf0, axis=0, keepdims=True)
    Mpf_ref[...] += lax.dot_general(pf0, pf0, (((0,), (0,)), ((), ())),
                                    preferred_element_type=F32, precision=lax.Precision.HIGHEST)


def _run_kD(fc, f0, S1, M1, Wr0, g0, b0, S2, M2, Wr1, g1, b1, S3, M3, Wv0, g2, b2):
    cmap = lambda i: (0, 0)
    return pl.pallas_call(
        _kD_body,
        grid=(_NBD,),
        in_specs=[pl.BlockSpec((_BD, 3), lambda i: (i, 0)),
                  pl.BlockSpec((_BD, 64), lambda i: (i, 0)),
                  pl.BlockSpec((1, 3), cmap), pl.BlockSpec((3, 3), cmap),
                  pl.BlockSpec((3, 16), cmap), pl.BlockSpec((1, 16), cmap),
                  pl.BlockSpec((1, 16), cmap), pl.BlockSpec((1, 16), cmap),
                  pl.BlockSpec((16, 16), cmap), pl.BlockSpec((16, 64), cmap),
                  pl.BlockSpec((1, 64), cmap), pl.BlockSpec((1, 64), cmap),
                  pl.BlockSpec((1, 64), cmap), pl.BlockSpec((64, 64), cmap),
                  pl.BlockSpec((64, 128), cmap), pl.BlockSpec((1, 128), cmap),
                  pl.BlockSpec((1, 128), cmap)],
        out_specs=[pl.BlockSpec((_BD, 128), lambda i: (i, 0)),
                   pl.BlockSpec((1, 128), cmap),
                   pl.BlockSpec((128, 128), cmap)],
        out_shape=[jax.ShapeDtypeStruct((N, 128), F32),
                   jax.ShapeDtypeStruct((1, 128), F32),
                   jax.ShapeDtypeStruct((128, 128), F32)],
        scratch_shapes=[pltpu.VMEM((2, 128), F32)],
    )(fc, f0, S1, M1, Wr0, g0, b0, S2, M2, Wr1, g1, b1, S3, M3, Wv0, g2, b2)


_RW = N // 32
_CH = 80
_NCH = _RW // _CH
_ZR = KP // 16


def _sc_sums_body(pf_hbm, inv_hbm, zer128, sums_hbm, idx_v, rows_v, accum):
    cid = lax.axis_index("c")
    sid = lax.axis_index("s")
    wid = sid * 2 + cid

    pltpu.sync_copy(zer128, accum.at[pl.ds(sid * _ZR, _ZR)])
    plsc.subcore_barrier()

    def body(t, carry):
        base = wid * _RW + t * _CH
        pltpu.sync_copy(inv_hbm.at[pl.ds(base, _CH)], idx_v)
        pltpu.sync_copy(pf_hbm.at[pl.ds(base, _CH)], rows_v)
        pltpu.sync_copy(rows_v, accum.at[idx_v], add=True)
        return carry

    lax.fori_loop(0, _NCH, body, 0)
    plsc.subcore_barrier()

    pltpu.sync_copy(accum.at[pl.ds(sid * _ZR, _ZR)],
                    sums_hbm.at[pl.ds(cid * KP + sid * _ZR, _ZR)])


def _seg_sums_sc(pf, inv_i32):
    mesh = plsc.VectorSubcoreMesh(core_axis_name="c", subcore_axis_name="s")
    run = pl.kernel(
        _sc_sums_body,
        mesh=mesh,
        out_type=[jax.ShapeDtypeStruct((2 * KP, 128), F32)],
        scratch_types=[pltpu.VMEM((_CH,), jnp.int32),
                       pltpu.VMEM((_CH, 128), F32),
                       pltpu.VMEM_SHARED((KP, 128), F32)],
    )
    (sums,) = run(pf, inv_i32, jnp.zeros((_ZR, 128), F32))
    return sums


def _sc_counts_body(inv_hbm, zer128, ones128, counts_hbm, idx_v, ones_v, accum):
    cid = lax.axis_index("c")
    sid = lax.axis_index("s")
    wid = sid * 2 + cid

    pltpu.sync_copy(zer128, accum.at[pl.ds(sid * _ZR, _ZR)])
    pltpu.sync_copy(ones128, ones_v)
    plsc.subcore_barrier()

    def body(t, carry):
        base = wid * _RW + t * _CH
        pltpu.sync_copy(inv_hbm.at[pl.ds(base, _CH)], idx_v)
        pltpu.sync_copy(ones_v, accum.at[idx_v], add=True)
        return carry

    lax.fori_loop(0, _NCH, body, 0)
    plsc.subcore_barrier()

    pltpu.sync_copy(accum.at[pl.ds(sid * _ZR, _ZR)],
                    counts_hbm.at[pl.ds(cid * KP + sid * _ZR, _ZR)])


def _seg_counts_sc(inv_i32):
    mesh = plsc.VectorSubcoreMesh(core_axis_name="c", subcore_axis_name="s")
    run = pl.kernel(
        _sc_counts_body,
        mesh=mesh,
        out_type=[jax.ShapeDtypeStruct((2 * KP, 128), F32)],
        scratch_types=[pltpu.VMEM((_CH,), jnp.int32),
                       pltpu.VMEM((_CH, 128), F32),
                       pltpu.VMEM_SHARED((KP, 128), F32)],
    )
    (counts,) = run(inv_i32, jnp.zeros((_ZR, 128), F32),
                    jnp.ones((_CH, 128), F32))
    return counts


_BK = 512
_NBK = KP // _BK


def _kM_body(s0a_ref, s0b_ref, cta_ref, ctb_ref, Wb_ref, u_ref, C_ref):
    i = pl.program_id(0)

    @pl.when(i == 0)
    def _():
        C_ref[...] = jnp.zeros_like(C_ref)

    s0 = s0a_ref[...] + s0b_ref[...]
    cnt = (cta_ref[...] + ctb_ref[...])[:, 0:1]
    vf0 = jnp.where(cnt > 0, s0 / cnt, 0.0)
    u_ref[...] = jnp.dot(vf0, Wb_ref[...], preferred_element_type=F32)
    C_ref[...] += lax.dot_general(s0, vf0, (((0,), (0,)), ((), ())),
                                  preferred_element_type=F32, precision=lax.Precision.HIGHEST)


def _run_kM(sums, counts, Wb):
    cmap = lambda i: (0, 0)
    return pl.pallas_call(
        _kM_body,
        grid=(_NBK,),
        in_specs=[pl.BlockSpec((_BK, 128), lambda i: (i, 0)),
                  pl.BlockSpec((_BK, 128), lambda i: (i + _NBK, 0)),
                  pl.BlockSpec((_BK, 128), lambda i: (i, 0)),
                  pl.BlockSpec((_BK, 128), lambda i: (i + _NBK, 0)),
                  pl.BlockSpec((128, 128), cmap)],
        out_specs=[pl.BlockSpec((_BK, 128), lambda i: (i, 0)),
                   pl.BlockSpec((128, 128), cmap)],
        out_shape=[jax.ShapeDtypeStruct((KP, 128), F32),
                   jax.ShapeDtypeStruct((128, 128), F32)],
    )(sums, sums, counts, counts, Wb)


_BE = 128
_NBE = N // _BE
_WIN = _BE + 8


def _kE_body(pf0_ref, inv_ref, u_ref, Wa_ref, Wb_ref, C_ref, Mpf_ref,
             Spf_ref, g3_ref, b3_ref, pf1_ref, aff_scr):
    i = pl.program_id(0)

    @pl.when(i == 0)
    def _():
        Wa = Wa_ref[...]
        Wb = Wb_ref[...]
        C = C_ref[...]
        mu3 = jnp.dot(Spf_ref[...], Wa + Wb, preferred_element_type=F32, precision=lax.Precision.HIGHEST) / N
        d1 = jnp.sum(Wa * jnp.dot(Mpf_ref[...], Wa, preferred_element_type=F32, precision=lax.Precision.HIGHEST), axis=0, keepdims=True)
        d2 = jnp.sum(Wa * jnp.dot(C, Wb, preferred_element_type=F32, precision=lax.Precision.HIGHEST), axis=0, keepdims=True)
        d3 = jnp.sum(Wb * jnp.dot(C, Wb, preferred_element_type=F32, precision=lax.Precision.HIGHEST), axis=0, keepdims=True)
        var3 = (d1 + 2.0 * d2 + d3) / N - mu3 * mu3
        a3 = g3_ref[...] / jnp.sqrt(var3 + 1e-5)
        aff_scr[0:1, :] = a3
        aff_scr[1:2, :] = b3_ref[...] - mu3 * a3

    inv = inv_ref[0]
    s_b = inv_ref[0, 0, 0]
    s8 = (s_b // 8) * 8
    u_win = u_ref[pl.ds(s8, _WIN), :]
    col = lax.broadcasted_iota(jnp.int32, (_BE, _WIN), 1)
    onehot = (col == (inv - s8)).astype(F32)
    g = jnp.dot(onehot, u_win, preferred_element_type=F32, precision=lax.Precision.HIGHEST)
    t3 = jnp.dot(pf0_ref[...], Wa_ref[...], preferred_element_type=F32) + g
    pf1_ref[...] = jnp.maximum(t3 * aff_scr[0:1, :] + aff_scr[1:2, :], 0.0)


def _run_kE(pf0, inv3, u, Wa, Wb, C, Mpf, Spf, g3, b3):
    cmap = lambda i: (0, 0)
    return pl.pallas_call(
        _kE_body,
        grid=(_NBE,),
        in_specs=[pl.BlockSpec((_BE, 128), lambda i: (i, 0)),
                  pl.BlockSpec((1, _BE, 1), lambda i: (i, 0, 0)),
                  pl.BlockSpec((KP, 128), cmap),
                  pl.BlockSpec((128, 128), cmap),
                  pl.BlockSpec((128, 128), cmap),
                  pl.BlockSpec((128, 128), cmap),
                  pl.BlockSpec((128, 128), cmap),
                  pl.BlockSpec((1, 128), cmap),
                  pl.BlockSpec((1, 128), cmap),
                  pl.BlockSpec((1, 128), cmap)],
        out_specs=pl.BlockSpec((_BE, 128), lambda i: (i, 0)),
        out_shape=jax.ShapeDtypeStruct((N, 128), F32),
        scratch_shapes=[pltpu.VMEM((2, 128), F32)],
    )(pf0, inv3, u, Wa, Wb, C, Mpf, Spf, g3, b3)


def _kF_body(s0a_ref, s0b_ref, s1a_ref, s1b_ref, cta_ref, ctb_ref,
             Wf0_ref, Wf1_ref, bf_ref, out_ref):
    cnt = (cta_ref[...] + ctb_ref[...])[:, 0:1]
    vf0 = (s0a_ref[...] + s0b_ref[...]) / cnt
    vf1 = (s1a_ref[...] + s1b_ref[...]) / cnt
    out_ref[...] = (jnp.dot(vf0, Wf0_ref[...], preferred_element_type=F32)
                    + jnp.dot(vf1, Wf1_ref[...], preferred_element_type=F32)
                    + bf_ref[...])


def _run_kF(sums0, sums1, counts, Wf0, Wf1, bf):
    cmap = lambda i: (0, 0)
    return pl.pallas_call(
        _kF_body,
        grid=(_NBK,),
        in_specs=[pl.BlockSpec((_BK, 128), lambda i: (i, 0)),
                  pl.BlockSpec((_BK, 128), lambda i: (i + _NBK, 0)),
                  pl.BlockSpec((_BK, 128), lambda i: (i, 0)),
                  pl.BlockSpec((_BK, 128), lambda i: (i + _NBK, 0)),
                  pl.BlockSpec((_BK, 128), lambda i: (i, 0)),
                  pl.BlockSpec((_BK, 128), lambda i: (i + _NBK, 0)),
                  pl.BlockSpec((128, 128), cmap),
                  pl.BlockSpec((128, 128), cmap),
                  pl.BlockSpec((1, 128), cmap)],
        out_specs=pl.BlockSpec((_BK, 128), lambda i: (i, 0)),
        out_shape=jax.ShapeDtypeStruct((KP, 128), F32),
    )(sums0, sums0, sums1, sums1, counts, counts, Wf0, Wf1, bf)


def kernel(coors, points, features, f_cluster, W_r0, g_r0, b_r0, W_r1, g_r1,
           b_r1, W_v0, g_v0, b_v0, W_v1, g_v1, b_v1, W_f, b_f):
    coors_i32 = coors.astype(jnp.int32)
    fc = f_cluster
    feat0 = jnp.concatenate([points, features], axis=1)
    r2 = lambda v: v.reshape(1, -1)

    inv = _compute_inv(coors_i32)

    S1, M1 = _run_kA(fc)
    S2, M2 = _run_kB(fc, S1, M1, W_r0, r2(g_r0), r2(b_r0))
    S3, M3 = _run_kC(fc, feat0, S1, M1, W_r0, r2(g_r0), r2(b_r0),
                     S2, M2, W_r1, r2(g_r1), r2(b_r1))
    pf0, Spf, Mpf = _run_kD(fc, feat0, S1, M1, W_r0, r2(g_r0), r2(b_r0),
                            S2, M2, W_r1, r2(g_r1), r2(b_r1),
                            S3, M3, W_v0, r2(g_v0), r2(b_v0))

    sums0 = _seg_sums_sc(pf0, inv)
    counts = _seg_counts_sc(inv)

    Wa = W_v1[:128]
    Wb = W_v1[128:]
    u, C = _run_kM(sums0, counts, Wb)

    inv3 = inv.reshape(_NBE, _BE, 1)
    pf1 = _run_kE(pf0, inv3, u, Wa, Wb, C, Mpf, Spf, r2(g_v1), r2(b_v1))

    sums1 = _seg_sums_sc(pf1, inv)

    vox = _run_kF(sums0, sums1, counts, W_f[:128], W_f[128:], r2(b_f))
    return pf1, vox[:K]


# speedup vs baseline: 1.5957x; 1.2165x over previous
import functools
import jax
import jax.numpy as jnp
from jax import lax
from jax.experimental import pallas as pl
from jax.experimental.pallas import tpu as pltpu
from jax.experimental.pallas import tpu_sc as plsc

N = 320000
K = 10000
KP = 10240
F32 = jnp.float32

_BI = 3200
_NBI = N // _BI


def _inv_body(coors_ref, inv_ref, carry):
    i = pl.program_id(0)
    blk = coors_ref[0]

    @pl.when(i == 0)
    def _():
        carry[0] = blk[0, 0]
        carry[1] = 0

    prev = carry[0]
    shifted = jnp.roll(blk, 1, axis=1)
    col = lax.broadcasted_iota(jnp.int32, blk.shape, 1)
    shifted = jnp.where(col == 0, prev, shifted)
    flags = (blk != shifted).astype(jnp.int32)
    x = flags
    sh = 1
    while sh < _BI:
        x = x + jnp.where(col >= sh, jnp.roll(x, sh, axis=1), 0)
        sh *= 2
    inv_ref[0] = carry[1] + x
    carry[1] = carry[1] + jnp.sum(flags)
    carry[0] = blk[0, _BI - 1]


def _compute_inv(coors_i32):
    c3 = coors_i32.reshape(_NBI, 1, _BI)
    out = pl.pallas_call(
        _inv_body,
        grid=(_NBI,),
        in_specs=[pl.BlockSpec((1, 1, _BI), lambda i: (i, 0, 0))],
        out_specs=pl.BlockSpec((1, 1, _BI), lambda i: (i, 0, 0)),
        out_shape=jax.ShapeDtypeStruct((_NBI, 1, _BI), jnp.int32),
        scratch_shapes=[pltpu.SMEM((2,), jnp.int32)],
    )(c3)
    return out.reshape(N)


def _bn_affine(S, M, W, g, b):
    mu = jnp.dot(S, W, preferred_element_type=F32, precision=lax.Precision.HIGHEST) / N
    ex2 = jnp.sum(W * jnp.dot(M, W, preferred_element_type=F32, precision=lax.Precision.HIGHEST), axis=0, keepdims=True) / N
    var = ex2 - mu * mu
    a = g / jnp.sqrt(var + 1e-5)
    c = b - mu * a
    return a, c


_BA = 2000
_NBA = N // _BA


def _kA_body(fc_ref, S1_ref, M1_ref):
    i = pl.program_id(0)

    @pl.when(i == 0)
    def _():
        S1_ref[...] = jnp.zeros_like(S1_ref)
        M1_ref[...] = jnp.zeros_like(M1_ref)

    fc = fc_ref[...]
    S1_ref[...] += jnp.sum(fc, axis=0, keepdims=True)
    M1_ref[...] += lax.dot_general(fc, fc, (((0,), (0,)), ((), ())),
                                   preferred_element_type=F32, precision=lax.Precision.HIGHEST)


def _run_kA(fc):
    return pl.pallas_call(
        _kA_body,
        grid=(_NBA,),
        in_specs=[pl.BlockSpec((_BA, 3), lambda i: (i, 0))],
        out_specs=[pl.BlockSpec((1, 3), lambda i: (0, 0)),
                   pl.BlockSpec((3, 3), lambda i: (0, 0))],
        out_shape=[jax.ShapeDtypeStruct((1, 3), F32),
                   jax.ShapeDtypeStruct((3, 3), F32)],
    )(fc)


def _kB_body(fc_ref, S1_ref, M1_ref, Wr0_ref, g0_ref, b0_ref, S2_ref, M2_ref):
    i = pl.program_id(0)

    @pl.when(i == 0)
    def _():
        S2_ref[...] = jnp.zeros_like(S2_ref)
        M2_ref[...] = jnp.zeros_like(M2_ref)

    Wr0 = Wr0_ref[...]
    a0, c0 = _bn_affine(S1_ref[...], M1_ref[...], Wr0, g0_ref[...], b0_ref[...])
    fc = fc_ref[...]
    h = jnp.maximum(jnp.dot(fc, Wr0, preferred_element_type=F32) * a0 + c0, 0.0)
    S2_ref[...] += jnp.sum(h, axis=0, keepdims=True)
    M2_ref[...] += lax.dot_general(h, h, (((0,), (0,)), ((), ())),
                                   preferred_element_type=F32, precision=lax.Precision.HIGHEST)


def _run_kB(fc, S1, M1, Wr0, g0, b0):
    return pl.pallas_call(
        _kB_body,
        grid=(_NBA,),
        in_specs=[pl.BlockSpec((_BA, 3), lambda i: (i, 0)),
                  pl.BlockSpec((1, 3), lambda i: (0, 0)),
                  pl.BlockSpec((3, 3), lambda i: (0, 0)),
                  pl.BlockSpec((3, 16), lambda i: (0, 0)),
                  pl.BlockSpec((1, 16), lambda i: (0, 0)),
                  pl.BlockSpec((1, 16), lambda i: (0, 0))],
        out_specs=[pl.BlockSpec((1, 16), lambda i: (0, 0)),
                   pl.BlockSpec((16, 16), lambda i: (0, 0))],
        out_shape=[jax.ShapeDtypeStruct((1, 16), F32),
                   jax.ShapeDtypeStruct((16, 16), F32)],
    )(fc, S1, M1, Wr0, g0, b0)


def _kC_body(fc_ref, f0_ref, S1_ref, M1_ref, Wr0_ref, g0_ref, b0_ref,
             S2_ref, M2_ref, Wr1_ref, g1_ref, b1_ref, S3_ref, M3_ref):
    i = pl.program_id(0)

    @pl.when(i == 0)
    def _():
        S3_ref[...] = jnp.zeros_like(S3_ref)
        M3_ref[...] = jnp.zeros_like(M3_ref)

    Wr0 = Wr0_ref[...]
    Wr1 = Wr1_ref[...]
    a0, c0 = _bn_affine(S1_ref[...], M1_ref[...], Wr0, g0_ref[...], b0_ref[...])
    a1, c1 = _bn_affine(S2_ref[...], M2_ref[...], Wr1, g1_ref[...], b1_ref[...])
    fc = fc_ref[...]
    h = jnp.maximum(jnp.dot(fc, Wr0, preferred_element_type=F32) * a0 + c0, 0.0)
    rel = jnp.maximum(jnp.dot(h, Wr1, preferred_element_type=F32) * a1 + c1, 0.0)
    feats = f0_ref[...] * rel
    S3_ref[...] += jnp.sum(feats, axis=0, keepdims=True)
    M3_ref[...] += lax.dot_general(feats, feats, (((0,), (0,)), ((), ())),
                                   preferred_element_type=F32, precision=lax.Precision.HIGHEST)


def _run_kC(fc, f0, S1, M1, Wr0, g0, b0, S2, M2, Wr1, g1, b1):
    cmap = lambda i: (0, 0)
    return pl.pallas_call(
        _kC_body,
        grid=(_NBA,),
        in_specs=[pl.BlockSpec((_BA, 3), lambda i: (i, 0)),
                  pl.BlockSpec((_BA, 64), lambda i: (i, 0)),
                  pl.BlockSpec((1, 3), cmap), pl.BlockSpec((3, 3), cmap),
                  pl.BlockSpec((3, 16), cmap), pl.BlockSpec((1, 16), cmap),
                  pl.BlockSpec((1, 16), cmap), pl.BlockSpec((1, 16), cmap),
                  pl.BlockSpec((16, 16), cmap), pl.BlockSpec((16, 64), cmap),
                  pl.BlockSpec((1, 64), cmap), pl.BlockSpec((1, 64), cmap)],
        out_specs=[pl.BlockSpec((1, 64), cmap),
                   pl.BlockSpec((64, 64), cmap)],
        out_shape=[jax.ShapeDtypeStruct((1, 64), F32),
                   jax.ShapeDtypeStruct((64, 64), F32)],
    )(fc, f0, S1, M1, Wr0, g0, b0, S2, M2, Wr1, g1, b1)


_BD = 1000
_NBD = N // _BD


def _kD_body(fc_ref, f0_ref, S1_ref, M1_ref, Wr0_ref, g0_ref, b0_ref,
             S2_ref, M2_ref, Wr1_ref, g1_ref, b1_ref,
             S3_ref, M3_ref, Wv0_ref, g2_ref, b2_ref,
             pf0_ref, Spf_ref, Mpf_ref, aff_scr):
    i = pl.program_id(0)

    @pl.when(i == 0)
    def _():
        Spf_ref[...] = jnp.zeros_like(Spf_ref)
        Mpf_ref[...] = jnp.zeros_like(Mpf_ref)
        a2, c2 = _bn_affine(S3_ref[...], M3_ref[...], Wv0_ref[...],
                            g2_ref[...], b2_ref[...])
        aff_scr[0:1, :] = a2
        aff_scr[1:2, :] = c2

    Wr0 = Wr0_ref[...]
    Wr1 = Wr1_ref[...]
    a0, c0 = _bn_affine(S1_ref[...], M1_ref[...], Wr0, g0_ref[...], b0_ref[...])
    a1, c1 = _bn_affine(S2_ref[...], M2_ref[...], Wr1, g1_ref[...], b1_ref[...])
    fc = fc_ref[...]
    h = jnp.maximum(jnp.dot(fc, Wr0, preferred_element_type=F32) * a0 + c0, 0.0)
    rel = jnp.maximum(jnp.dot(h, Wr1, preferred_element_type=F32) * a1 + c1, 0.0)
    feats = f0_ref[...] * rel
    t2 = jnp.dot(feats, Wv0_ref[...], preferred_element_type=F32)
    pf0 = jnp.maximum(t2 * aff_scr[0:1, :] + aff_scr[1:2, :], 0.0)
    pf0_ref[...] = pf0
    Spf_ref[...] += jnp.sum(pf0, axis=0, keepdims=True)
    Mpf_ref[...] += lax.dot_general(pf0, pf0, (((0,), (0,)), ((), ())),
                                    preferred_element_type=F32, precision=lax.Precision.HIGHEST)


def _run_kD(fc, f0, S1, M1, Wr0, g0, b0, S2, M2, Wr1, g1, b1, S3, M3, Wv0, g2, b2):
    cmap = lambda i: (0, 0)
    return pl.pallas_call(
        _kD_body,
        grid=(_NBD,),
        in_specs=[pl.BlockSpec((_BD, 3), lambda i: (i, 0)),
                  pl.BlockSpec((_BD, 64), lambda i: (i, 0)),
                  pl.BlockSpec((1, 3), cmap), pl.BlockSpec((3, 3), cmap),
                  pl.BlockSpec((3, 16), cmap), pl.BlockSpec((1, 16), cmap),
                  pl.BlockSpec((1, 16), cmap), pl.BlockSpec((1, 16), cmap),
                  pl.BlockSpec((16, 16), cmap), pl.BlockSpec((16, 64), cmap),
                  pl.BlockSpec((1, 64), cmap), pl.BlockSpec((1, 64), cmap),
                  pl.BlockSpec((1, 64), cmap), pl.BlockSpec((64, 64), cmap),
                  pl.BlockSpec((64, 128), cmap), pl.BlockSpec((1, 128), cmap),
                  pl.BlockSpec((1, 128), cmap)],
        out_specs=[pl.BlockSpec((_BD, 128), lambda i: (i, 0)),
                   pl.BlockSpec((1, 128), cmap),
                   pl.BlockSpec((128, 128), cmap)],
        out_shape=[jax.ShapeDtypeStruct((N, 128), F32),
                   jax.ShapeDtypeStruct((1, 128), F32),
                   jax.ShapeDtypeStruct((128, 128), F32)],
        scratch_shapes=[pltpu.VMEM((2, 128), F32)],
    )(fc, f0, S1, M1, Wr0, g0, b0, S2, M2, Wr1, g1, b1, S3, M3, Wv0, g2, b2)


_RW = N // 32
_CH = 80
_NCH = _RW // _CH
_ZR = KP // 16


def _sc_sums_body(pf_hbm, inv_hbm, zer128, sums_hbm, idx_v, rows_v, accum):
    cid = lax.axis_index("c")
    sid = lax.axis_index("s")
    wid = sid * 2 + cid

    pltpu.sync_copy(zer128, accum.at[pl.ds(sid * _ZR, _ZR)])
    plsc.subcore_barrier()

    def body(t, carry):
        base = wid * _RW + t * _CH
        pltpu.sync_copy(inv_hbm.at[pl.ds(base, _CH)], idx_v)
        pltpu.sync_copy(pf_hbm.at[pl.ds(base, _CH)], rows_v)
        pltpu.sync_copy(rows_v, accum.at[idx_v], add=True)
        return carry

    lax.fori_loop(0, _NCH, body, 0)
    plsc.subcore_barrier()

    pltpu.sync_copy(accum.at[pl.ds(sid * _ZR, _ZR)],
                    sums_hbm.at[pl.ds(cid * KP + sid * _ZR, _ZR)])


def _seg_sums_sc(pf, inv_i32):
    mesh = plsc.VectorSubcoreMesh(core_axis_name="c", subcore_axis_name="s")
    run = pl.kernel(
        _sc_sums_body,
        mesh=mesh,
        out_type=[jax.ShapeDtypeStruct((2 * KP, 128), F32)],
        scratch_types=[pltpu.VMEM((_CH,), jnp.int32),
                       pltpu.VMEM((_CH, 128), F32),
                       pltpu.VMEM_SHARED((KP, 128), F32)],
    )
    (sums,) = run(pf, inv_i32, jnp.zeros((_ZR, 128), F32))
    return sums


def _sc_counts_body(inv_hbm, zer128, ones128, counts_hbm, idx_v, ones_v, accum):
    cid = lax.axis_index("c")
    sid = lax.axis_index("s")
    wid = sid * 2 + cid

    pltpu.sync_copy(zer128, accum.at[pl.ds(sid * _ZR, _ZR)])
    pltpu.sync_copy(ones128, ones_v)
    plsc.subcore_barrier()

    def body(t, carry):
        base = wid * _RW + t * _CH
        pltpu.sync_copy(inv_hbm.at[pl.ds(base, _CH)], idx_v)
        pltpu.sync_copy(ones_v, accum.at[idx_v], add=True)
        return carry

    lax.fori_loop(0, _NCH, body, 0)
    plsc.subcore_barrier()

    pltpu.sync_copy(accum.at[pl.ds(sid * _ZR, _ZR)],
                    counts_hbm.at[pl.ds(cid * KP + sid * _ZR, _ZR)])


def _sc_gather_body(u_hbm, inv_hbm, g_hbm, idx_v, rows_v, sem):
    cid = lax.axis_index("c")
    sid = lax.axis_index("s")
    wid = sid * 2 + cid

    def body(t, carry):
        base = wid * _RW + t * _CH
        pltpu.sync_copy(inv_hbm.at[pl.ds(base, _CH)], idx_v)
        pltpu.async_copy(u_hbm.at[idx_v], rows_v, sem).wait()
        pltpu.sync_copy(rows_v, g_hbm.at[pl.ds(base, _CH)])
        return carry

    lax.fori_loop(0, _NCH, body, 0)


def _gather_sc(u, inv_i32):
    mesh = plsc.VectorSubcoreMesh(core_axis_name="c", subcore_axis_name="s")
    run = pl.kernel(
        _sc_gather_body,
        mesh=mesh,
        out_type=[jax.ShapeDtypeStruct((N, 128), F32)],
        scratch_types=[pltpu.VMEM((_CH,), jnp.int32),
                       pltpu.VMEM((_CH, 128), F32),
                       pltpu.SemaphoreType.DMA],
    )
    (g,) = run(u, inv_i32)
    return g


def _seg_counts_sc(inv_i32):
    mesh = plsc.VectorSubcoreMesh(core_axis_name="c", subcore_axis_name="s")
    run = pl.kernel(
        _sc_counts_body,
        mesh=mesh,
        out_type=[jax.ShapeDtypeStruct((2 * KP, 128), F32)],
        scratch_types=[pltpu.VMEM((_CH,), jnp.int32),
                       pltpu.VMEM((_CH, 128), F32),
                       pltpu.VMEM_SHARED((KP, 128), F32)],
    )
    (counts,) = run(inv_i32, jnp.zeros((_ZR, 128), F32),
                    jnp.ones((_CH, 128), F32))
    return counts


_BK = 512
_NBK = KP // _BK


def _kM_body(s0a_ref, s0b_ref, cta_ref, ctb_ref, Wb_ref, u_ref, C_ref):
    i = pl.program_id(0)

    @pl.when(i == 0)
    def _():
        C_ref[...] = jnp.zeros_like(C_ref)

    s0 = s0a_ref[...] + s0b_ref[...]
    cnt = (cta_ref[...] + ctb_ref[...])[:, 0:1]
    vf0 = jnp.where(cnt > 0, s0 / cnt, 0.0)
    u_ref[...] = jnp.dot(vf0, Wb_ref[...], preferred_element_type=F32)
    C_ref[...] += lax.dot_general(s0, vf0, (((0,), (0,)), ((), ())),
                                  preferred_element_type=F32, precision=lax.Precision.HIGHEST)


def _run_kM(sums, counts, Wb):
    cmap = lambda i: (0, 0)
    return pl.pallas_call(
        _kM_body,
        grid=(_NBK,),
        in_specs=[pl.BlockSpec((_BK, 128), lambda i: (i, 0)),
                  pl.BlockSpec((_BK, 128), lambda i: (i + _NBK, 0)),
                  pl.BlockSpec((_BK, 128), lambda i: (i, 0)),
                  pl.BlockSpec((_BK, 128), lambda i: (i + _NBK, 0)),
                  pl.BlockSpec((128, 128), cmap)],
        out_specs=[pl.BlockSpec((_BK, 128), lambda i: (i, 0)),
                   pl.BlockSpec((128, 128), cmap)],
        out_shape=[jax.ShapeDtypeStruct((KP, 128), F32),
                   jax.ShapeDtypeStruct((128, 128), F32)],
    )(sums, sums, counts, counts, Wb)


_BE = 512
_NBE = N // _BE


def _kE_body(pf0_ref, g_ref, Wa_ref, Wb_ref, C_ref, Mpf_ref,
             Spf_ref, g3_ref, b3_ref, pf1_ref, aff_scr):
    i = pl.program_id(0)

    @pl.when(i == 0)
    def _():
        Wa = Wa_ref[...]
        Wb = Wb_ref[...]
        C = C_ref[...]
        mu3 = jnp.dot(Spf_ref[...], Wa + Wb, preferred_element_type=F32, precision=lax.Precision.HIGHEST) / N
        d1 = jnp.sum(Wa * jnp.dot(Mpf_ref[...], Wa, preferred_element_type=F32, precision=lax.Precision.HIGHEST), axis=0, keepdims=True)
        d2 = jnp.sum(Wa * jnp.dot(C, Wb, preferred_element_type=F32, precision=lax.Precision.HIGHEST), axis=0, keepdims=True)
        d3 = jnp.sum(Wb * jnp.dot(C, Wb, preferred_element_type=F32, precision=lax.Precision.HIGHEST), axis=0, keepdims=True)
        var3 = (d1 + 2.0 * d2 + d3) / N - mu3 * mu3
        a3 = g3_ref[...] / jnp.sqrt(var3 + 1e-5)
        aff_scr[0:1, :] = a3
        aff_scr[1:2, :] = b3_ref[...] - mu3 * a3

    t3 = jnp.dot(pf0_ref[...], Wa_ref[...], preferred_element_type=F32) + g_ref[...]
    pf1_ref[...] = jnp.maximum(t3 * aff_scr[0:1, :] + aff_scr[1:2, :], 0.0)


def _run_kE(pf0, g, Wa, Wb, C, Mpf, Spf, g3, b3):
    cmap = lambda i: (0, 0)
    return pl.pallas_call(
        _kE_body,
        grid=(_NBE,),
        in_specs=[pl.BlockSpec((_BE, 128), lambda i: (i, 0)),
                  pl.BlockSpec((_BE, 128), lambda i: (i, 0)),
                  pl.BlockSpec((128, 128), cmap),
                  pl.BlockSpec((128, 128), cmap),
                  pl.BlockSpec((128, 128), cmap),
                  pl.BlockSpec((128, 128), cmap),
                  pl.BlockSpec((1, 128), cmap),
                  pl.BlockSpec((1, 128), cmap),
                  pl.BlockSpec((1, 128), cmap)],
        out_specs=pl.BlockSpec((_BE, 128), lambda i: (i, 0)),
        out_shape=jax.ShapeDtypeStruct((N, 128), F32),
        scratch_shapes=[pltpu.VMEM((2, 128), F32)],
    )(pf0, g, Wa, Wb, C, Mpf, Spf, g3, b3)


def _kF_body(s0a_ref, s0b_ref, s1a_ref, s1b_ref, cta_ref, ctb_ref,
             Wf0_ref, Wf1_ref, bf_ref, out_ref):
    cnt = (cta_ref[...] + ctb_ref[...])[:, 0:1]
    vf0 = (s0a_ref[...] + s0b_ref[...]) / cnt
    vf1 = (s1a_ref[...] + s1b_ref[...]) / cnt
    out_ref[...] = (jnp.dot(vf0, Wf0_ref[...], preferred_element_type=F32)
                    + jnp.dot(vf1, Wf1_ref[...], preferred_element_type=F32)
                    + bf_ref[...])


def _run_kF(sums0, sums1, counts, Wf0, Wf1, bf):
    cmap = lambda i: (0, 0)
    return pl.pallas_call(
        _kF_body,
        grid=(_NBK,),
        in_specs=[pl.BlockSpec((_BK, 128), lambda i: (i, 0)),
                  pl.BlockSpec((_BK, 128), lambda i: (i + _NBK, 0)),
                  pl.BlockSpec((_BK, 128), lambda i: (i, 0)),
                  pl.BlockSpec((_BK, 128), lambda i: (i + _NBK, 0)),
                  pl.BlockSpec((_BK, 128), lambda i: (i, 0)),
                  pl.BlockSpec((_BK, 128), lambda i: (i + _NBK, 0)),
                  pl.BlockSpec((128, 128), cmap),
                  pl.BlockSpec((128, 128), cmap),
                  pl.BlockSpec((1, 128), cmap)],
        out_specs=pl.BlockSpec((_BK, 128), lambda i: (i, 0)),
        out_shape=jax.ShapeDtypeStruct((KP, 128), F32),
    )(sums0, sums0, sums1, sums1, counts, counts, Wf0, Wf1, bf)


def kernel(coors, points, features, f_cluster, W_r0, g_r0, b_r0, W_r1, g_r1,
           b_r1, W_v0, g_v0, b_v0, W_v1, g_v1, b_v1, W_f, b_f):
    coors_i32 = coors.astype(jnp.int32)
    fc = f_cluster
    feat0 = jnp.concatenate([points, features], axis=1)
    r2 = lambda v: v.reshape(1, -1)

    inv = _compute_inv(coors_i32)

    S1, M1 = _run_kA(fc)
    S2, M2 = _run_kB(fc, S1, M1, W_r0, r2(g_r0), r2(b_r0))
    S3, M3 = _run_kC(fc, feat0, S1, M1, W_r0, r2(g_r0), r2(b_r0),
                     S2, M2, W_r1, r2(g_r1), r2(b_r1))
    pf0, Spf, Mpf = _run_kD(fc, feat0, S1, M1, W_r0, r2(g_r0), r2(b_r0),
                            S2, M2, W_r1, r2(g_r1), r2(b_r1),
                            S3, M3, W_v0, r2(g_v0), r2(b_v0))

    sums0 = _seg_sums_sc(pf0, inv)
    counts = _seg_counts_sc(inv)

    Wa = W_v1[:128]
    Wb = W_v1[128:]
    u, C = _run_kM(sums0, counts, Wb)

    g = _gather_sc(u, inv)
    pf1 = _run_kE(pf0, g, Wa, Wb, C, Mpf, Spf, r2(g_v1), r2(b_v1))

    sums1 = _seg_sums_sc(pf1, inv)

    vox = _run_kF(sums0, sums1, counts, W_f[:128], W_f[128:], r2(b_f))
    return pf1, vox[:K]


# speedup vs baseline: 1.6798x; 1.0527x over previous
import functools
import jax
import jax.numpy as jnp
from jax import lax
from jax.experimental import pallas as pl
from jax.experimental.pallas import tpu as pltpu
from jax.experimental.pallas import tpu_sc as plsc

N = 320000
K = 10000
KP = 10240
F32 = jnp.float32

_BI = 3200
_NBI = N // _BI


def _inv_body(coors_ref, inv_ref, carry):
    i = pl.program_id(0)
    blk = coors_ref[0]

    @pl.when(i == 0)
    def _():
        carry[0] = blk[0, 0]
        carry[1] = 0

    prev = carry[0]
    shifted = jnp.roll(blk, 1, axis=1)
    col = lax.broadcasted_iota(jnp.int32, blk.shape, 1)
    shifted = jnp.where(col == 0, prev, shifted)
    flags = (blk != shifted).astype(jnp.int32)
    x = flags
    sh = 1
    while sh < _BI:
        x = x + jnp.where(col >= sh, jnp.roll(x, sh, axis=1), 0)
        sh *= 2
    inv_ref[0] = carry[1] + x
    carry[1] = carry[1] + jnp.sum(flags)
    carry[0] = blk[0, _BI - 1]


def _compute_inv(coors_i32):
    c3 = coors_i32.reshape(_NBI, 1, _BI)
    out = pl.pallas_call(
        _inv_body,
        grid=(_NBI,),
        in_specs=[pl.BlockSpec((1, 1, _BI), lambda i: (i, 0, 0))],
        out_specs=pl.BlockSpec((1, 1, _BI), lambda i: (i, 0, 0)),
        out_shape=jax.ShapeDtypeStruct((_NBI, 1, _BI), jnp.int32),
        scratch_shapes=[pltpu.SMEM((2,), jnp.int32)],
    )(c3)
    return out.reshape(N)


def _bn_affine(S, M, W, g, b):
    mu = jnp.dot(S, W, preferred_element_type=F32, precision=lax.Precision.HIGHEST) / N
    ex2 = jnp.sum(W * jnp.dot(M, W, preferred_element_type=F32, precision=lax.Precision.HIGHEST), axis=0, keepdims=True) / N
    var = ex2 - mu * mu
    a = g / jnp.sqrt(var + 1e-5)
    c = b - mu * a
    return a, c


_BA = 2000
_NBA = N // _BA


def _kA_body(fc_ref, S1_ref, M1_ref):
    i = pl.program_id(0)

    @pl.when(i == 0)
    def _():
        S1_ref[...] = jnp.zeros_like(S1_ref)
        M1_ref[...] = jnp.zeros_like(M1_ref)

    fc = fc_ref[...]
    S1_ref[...] += jnp.sum(fc, axis=0, keepdims=True)
    M1_ref[...] += lax.dot_general(fc, fc, (((0,), (0,)), ((), ())),
                                   preferred_element_type=F32, precision=lax.Precision.HIGHEST)


def _run_kA(fc):
    return pl.pallas_call(
        _kA_body,
        grid=(_NBA,),
        in_specs=[pl.BlockSpec((_BA, 3), lambda i: (i, 0))],
        out_specs=[pl.BlockSpec((1, 3), lambda i: (0, 0)),
                   pl.BlockSpec((3, 3), lambda i: (0, 0))],
        out_shape=[jax.ShapeDtypeStruct((1, 3), F32),
                   jax.ShapeDtypeStruct((3, 3), F32)],
    )(fc)


def _kB_body(fc_ref, S1_ref, M1_ref, Wr0_ref, g0_ref, b0_ref, S2_ref, M2_ref):
    i = pl.program_id(0)

    @pl.when(i == 0)
    def _():
        S2_ref[...] = jnp.zeros_like(S2_ref)
        M2_ref[...] = jnp.zeros_like(M2_ref)

    Wr0 = Wr0_ref[...]
    a0, c0 = _bn_affine(S1_ref[...], M1_ref[...], Wr0, g0_ref[...], b0_ref[...])
    fc = fc_ref[...]
    h = jnp.maximum(jnp.dot(fc, Wr0, preferred_element_type=F32) * a0 + c0, 0.0)
    S2_ref[...] += jnp.sum(h, axis=0, keepdims=True)
    M2_ref[...] += lax.dot_general(h, h, (((0,), (0,)), ((), ())),
                                   preferred_element_type=F32, precision=lax.Precision.HIGHEST)


def _run_kB(fc, S1, M1, Wr0, g0, b0):
    return pl.pallas_call(
        _kB_body,
        grid=(_NBA,),
        in_specs=[pl.BlockSpec((_BA, 3), lambda i: (i, 0)),
                  pl.BlockSpec((1, 3), lambda i: (0, 0)),
                  pl.BlockSpec((3, 3), lambda i: (0, 0)),
                  pl.BlockSpec((3, 16), lambda i: (0, 0)),
                  pl.BlockSpec((1, 16), lambda i: (0, 0)),
                  pl.BlockSpec((1, 16), lambda i: (0, 0))],
        out_specs=[pl.BlockSpec((1, 16), lambda i: (0, 0)),
                   pl.BlockSpec((16, 16), lambda i: (0, 0))],
        out_shape=[jax.ShapeDtypeStruct((1, 16), F32),
                   jax.ShapeDtypeStruct((16, 16), F32)],
    )(fc, S1, M1, Wr0, g0, b0)


def _kC_body(fc_ref, f0_ref, S1_ref, M1_ref, Wr0_ref, g0_ref, b0_ref,
             S2_ref, M2_ref, Wr1_ref, g1_ref, b1_ref, S3_ref, M3_ref):
    i = pl.program_id(0)

    @pl.when(i == 0)
    def _():
        S3_ref[...] = jnp.zeros_like(S3_ref)
        M3_ref[...] = jnp.zeros_like(M3_ref)

    Wr0 = Wr0_ref[...]
    Wr1 = Wr1_ref[...]
    a0, c0 = _bn_affine(S1_ref[...], M1_ref[...], Wr0, g0_ref[...], b0_ref[...])
    a1, c1 = _bn_affine(S2_ref[...], M2_ref[...], Wr1, g1_ref[...], b1_ref[...])
    fc = fc_ref[...]
    h = jnp.maximum(jnp.dot(fc, Wr0, preferred_element_type=F32) * a0 + c0, 0.0)
    rel = jnp.maximum(jnp.dot(h, Wr1, preferred_element_type=F32) * a1 + c1, 0.0)
    feats = f0_ref[...] * rel
    S3_ref[...] += jnp.sum(feats, axis=0, keepdims=True)
    M3_ref[...] += lax.dot_general(feats, feats, (((0,), (0,)), ((), ())),
                                   preferred_element_type=F32, precision=lax.Precision.HIGHEST)


def _run_kC(fc, f0, S1, M1, Wr0, g0, b0, S2, M2, Wr1, g1, b1):
    cmap = lambda i: (0, 0)
    return pl.pallas_call(
        _kC_body,
        grid=(_NBA,),
        in_specs=[pl.BlockSpec((_BA, 3), lambda i: (i, 0)),
                  pl.BlockSpec((_BA, 64), lambda i: (i, 0)),
                  pl.BlockSpec((1, 3), cmap), pl.BlockSpec((3, 3), cmap),
                  pl.BlockSpec((3, 16), cmap), pl.BlockSpec((1, 16), cmap),
                  pl.BlockSpec((1, 16), cmap), pl.BlockSpec((1, 16), cmap),
                  pl.BlockSpec((16, 16), cmap), pl.BlockSpec((16, 64), cmap),
                  pl.BlockSpec((1, 64), cmap), pl.BlockSpec((1, 64), cmap)],
        out_specs=[pl.BlockSpec((1, 64), cmap),
                   pl.BlockSpec((64, 64), cmap)],
        out_shape=[jax.ShapeDtypeStruct((1, 64), F32),
                   jax.ShapeDtypeStruct((64, 64), F32)],
    )(fc, f0, S1, M1, Wr0, g0, b0, S2, M2, Wr1, g1, b1)


_BD = 1000
_NBD = N // _BD


def _kD_body(fc_ref, f0_ref, S1_ref, M1_ref, Wr0_ref, g0_ref, b0_ref,
             S2_ref, M2_ref, Wr1_ref, g1_ref, b1_ref,
             S3_ref, M3_ref, Wv0_ref, g2_ref, b2_ref,
             pf0_ref, Spf_ref, Mpf_ref, aff_scr):
    i = pl.program_id(0)

    @pl.when(i == 0)
    def _():
        Spf_ref[...] = jnp.zeros_like(Spf_ref)
        Mpf_ref[...] = jnp.zeros_like(Mpf_ref)
        a2, c2 = _bn_affine(S3_ref[...], M3_ref[...], Wv0_ref[...],
                            g2_ref[...], b2_ref[...])
        aff_scr[0:1, :] = a2
        aff_scr[1:2, :] = c2

    Wr0 = Wr0_ref[...]
    Wr1 = Wr1_ref[...]
    a0, c0 = _bn_affine(S1_ref[...], M1_ref[...], Wr0, g0_ref[...], b0_ref[...])
    a1, c1 = _bn_affine(S2_ref[...], M2_ref[...], Wr1, g1_ref[...], b1_ref[...])
    fc = fc_ref[...]
    h = jnp.maximum(jnp.dot(fc, Wr0, preferred_element_type=F32) * a0 + c0, 0.0)
    rel = jnp.maximum(jnp.dot(h, Wr1, preferred_element_type=F32) * a1 + c1, 0.0)
    feats = f0_ref[...] * rel
    t2 = jnp.dot(feats, Wv0_ref[...], preferred_element_type=F32)
    pf0 = jnp.maximum(t2 * aff_scr[0:1, :] + aff_scr[1:2, :], 0.0)
    pf0_ref[...] = pf0
    Spf_ref[...] += jnp.sum(pf0, axis=0, keepdims=True)
    Mpf_ref[...] += lax.dot_general(pf0, pf0, (((0,), (0,)), ((), ())),
                                    preferred_element_type=F32, precision=lax.Precision.HIGHEST)


def _run_kD(fc, f0, S1, M1, Wr0, g0, b0, S2, M2, Wr1, g1, b1, S3, M3, Wv0, g2, b2):
    cmap = lambda i: (0, 0)
    return pl.pallas_call(
        _kD_body,
        grid=(_NBD,),
        in_specs=[pl.BlockSpec((_BD, 3), lambda i: (i, 0)),
                  pl.BlockSpec((_BD, 64), lambda i: (i, 0)),
                  pl.BlockSpec((1, 3), cmap), pl.BlockSpec((3, 3), cmap),
                  pl.BlockSpec((3, 16), cmap), pl.BlockSpec((1, 16), cmap),
                  pl.BlockSpec((1, 16), cmap), pl.BlockSpec((1, 16), cmap),
                  pl.BlockSpec((16, 16), cmap), pl.BlockSpec((16, 64), cmap),
                  pl.BlockSpec((1, 64), cmap), pl.BlockSpec((1, 64), cmap),
                  pl.BlockSpec((1, 64), cmap), pl.BlockSpec((64, 64), cmap),
                  pl.BlockSpec((64, 128), cmap), pl.BlockSpec((1, 128), cmap),
                  pl.BlockSpec((1, 128), cmap)],
        out_specs=[pl.BlockSpec((_BD, 128), lambda i: (i, 0)),
                   pl.BlockSpec((1, 128), cmap),
                   pl.BlockSpec((128, 128), cmap)],
        out_shape=[jax.ShapeDtypeStruct((N, 128), F32),
                   jax.ShapeDtypeStruct((1, 128), F32),
                   jax.ShapeDtypeStruct((128, 128), F32)],
        scratch_shapes=[pltpu.VMEM((2, 128), F32)],
    )(fc, f0, S1, M1, Wr0, g0, b0, S2, M2, Wr1, g1, b1, S3, M3, Wv0, g2, b2)


_RW = N // 32
_CH = 80
_NCH = _RW // _CH
_ZR = KP // 16


def _sc_sums_body(pf_hbm, inv_hbm, zer128, sums_hbm, idx_v, rows_v, accum):
    cid = lax.axis_index("c")
    sid = lax.axis_index("s")
    wid = sid * 2 + cid

    pltpu.sync_copy(zer128, accum.at[pl.ds(sid * _ZR, _ZR)])
    plsc.subcore_barrier()

    def body(t, carry):
        base = wid * _RW + t * _CH
        pltpu.sync_copy(inv_hbm.at[pl.ds(base, _CH)], idx_v)
        pltpu.sync_copy(pf_hbm.at[pl.ds(base, _CH)], rows_v)
        pltpu.sync_copy(rows_v, accum.at[idx_v], add=True)
        return carry

    lax.fori_loop(0, _NCH, body, 0)
    plsc.subcore_barrier()

    pltpu.sync_copy(accum.at[pl.ds(sid * _ZR, _ZR)],
                    sums_hbm.at[pl.ds(cid * KP + sid * _ZR, _ZR)])


def _seg_sums_sc(pf, inv_i32):
    mesh = plsc.VectorSubcoreMesh(core_axis_name="c", subcore_axis_name="s")
    run = pl.kernel(
        _sc_sums_body,
        mesh=mesh,
        out_type=[jax.ShapeDtypeStruct((2 * KP, 128), F32)],
        scratch_types=[pltpu.VMEM((_CH,), jnp.int32),
                       pltpu.VMEM((_CH, 128), F32),
                       pltpu.VMEM_SHARED((KP, 128), F32)],
    )
    (sums,) = run(pf, inv_i32, jnp.zeros((_ZR, 128), F32))
    return sums


def _sc_counts_body(inv_hbm, zer128, ones128, counts_hbm, idx_v, ones_v, accum):
    cid = lax.axis_index("c")
    sid = lax.axis_index("s")
    wid = sid * 2 + cid

    pltpu.sync_copy(zer128, accum.at[pl.ds(sid * _ZR, _ZR)])
    pltpu.sync_copy(ones128, ones_v)
    plsc.subcore_barrier()

    def body(t, carry):
        base = wid * _RW + t * _CH
        pltpu.sync_copy(inv_hbm.at[pl.ds(base, _CH)], idx_v)
        pltpu.sync_copy(ones_v, accum.at[idx_v], add=True)
        return carry

    lax.fori_loop(0, _NCH, body, 0)
    plsc.subcore_barrier()

    pltpu.sync_copy(accum.at[pl.ds(sid * _ZR, _ZR)],
                    counts_hbm.at[pl.ds(cid * KP + sid * _ZR, _ZR)])


def _sc_gather_body(u_hbm, inv_hbm, g_hbm, idx_a, idx_b, rows_a, rows_b,
                    sem_a, sem_b):
    cid = lax.axis_index("c")
    sid = lax.axis_index("s")
    wid = sid * 2 + cid
    base0 = wid * _RW

    def pair(p, carry):
        ba = base0 + (2 * p) * _CH
        bb = base0 + (2 * p + 1) * _CH
        pltpu.sync_copy(inv_hbm.at[pl.ds(ba, _CH)], idx_a)
        ga = pltpu.async_copy(u_hbm.at[idx_a], rows_a, sem_a)
        pltpu.sync_copy(inv_hbm.at[pl.ds(bb, _CH)], idx_b)
        gb = pltpu.async_copy(u_hbm.at[idx_b], rows_b, sem_b)
        ga.wait()
        pltpu.sync_copy(rows_a, g_hbm.at[pl.ds(ba, _CH)])
        gb.wait()
        pltpu.sync_copy(rows_b, g_hbm.at[pl.ds(bb, _CH)])
        return carry

    lax.fori_loop(0, _NCH // 2, pair, 0)
    bt = base0 + (_NCH - 1) * _CH
    pltpu.sync_copy(inv_hbm.at[pl.ds(bt, _CH)], idx_a)
    pltpu.async_copy(u_hbm.at[idx_a], rows_a, sem_a).wait()
    pltpu.sync_copy(rows_a, g_hbm.at[pl.ds(bt, _CH)])


def _gather_sc(u, inv_i32):
    mesh = plsc.VectorSubcoreMesh(core_axis_name="c", subcore_axis_name="s")
    run = pl.kernel(
        _sc_gather_body,
        mesh=mesh,
        out_type=[jax.ShapeDtypeStruct((N, 128), F32)],
        scratch_types=[pltpu.VMEM((_CH,), jnp.int32),
                       pltpu.VMEM((_CH,), jnp.int32),
                       pltpu.VMEM((_CH, 128), F32),
                       pltpu.VMEM((_CH, 128), F32),
                       pltpu.SemaphoreType.DMA,
                       pltpu.SemaphoreType.DMA],
    )
    (g,) = run(u, inv_i32)
    return g


def _seg_counts_sc(inv_i32):
    mesh = plsc.VectorSubcoreMesh(core_axis_name="c", subcore_axis_name="s")
    run = pl.kernel(
        _sc_counts_body,
        mesh=mesh,
        out_type=[jax.ShapeDtypeStruct((2 * KP, 128), F32)],
        scratch_types=[pltpu.VMEM((_CH,), jnp.int32),
                       pltpu.VMEM((_CH, 128), F32),
                       pltpu.VMEM_SHARED((KP, 128), F32)],
    )
    (counts,) = run(inv_i32, jnp.zeros((_ZR, 128), F32),
                    jnp.ones((_CH, 128), F32))
    return counts


_BK = 512
_NBK = KP // _BK


def _kM_body(s0a_ref, s0b_ref, cta_ref, ctb_ref, Wb_ref, u_ref, C_ref):
    i = pl.program_id(0)

    @pl.when(i == 0)
    def _():
        C_ref[...] = jnp.zeros_like(C_ref)

    s0 = s0a_ref[...] + s0b_ref[...]
    cnt = (cta_ref[...] + ctb_ref[...])[:, 0:1]
    vf0 = jnp.where(cnt > 0, s0 / cnt, 0.0)
    u_ref[...] = jnp.dot(vf0, Wb_ref[...], preferred_element_type=F32)
    C_ref[...] += lax.dot_general(s0, vf0, (((0,), (0,)), ((), ())),
                                  preferred_element_type=F32, precision=lax.Precision.HIGHEST)


def _run_kM(sums, counts, Wb):
    cmap = lambda i: (0, 0)
    return pl.pallas_call(
        _kM_body,
        grid=(_NBK,),
        in_specs=[pl.BlockSpec((_BK, 128), lambda i: (i, 0)),
                  pl.BlockSpec((_BK, 128), lambda i: (i + _NBK, 0)),
                  pl.BlockSpec((_BK, 128), lambda i: (i, 0)),
                  pl.BlockSpec((_BK, 128), lambda i: (i + _NBK, 0)),
                  pl.BlockSpec((128, 128), cmap)],
        out_specs=[pl.BlockSpec((_BK, 128), lambda i: (i, 0)),
                   pl.BlockSpec((128, 128), cmap)],
        out_shape=[jax.ShapeDtypeStruct((KP, 128), F32),
                   jax.ShapeDtypeStruct((128, 128), F32)],
    )(sums, sums, counts, counts, Wb)


_BE = 512
_NBE = N // _BE


def _kE_body(pf0_ref, g_ref, Wa_ref, Wb_ref, C_ref, Mpf_ref,
             Spf_ref, g3_ref, b3_ref, pf1_ref, aff_scr):
    i = pl.program_id(0)

    @pl.when(i == 0)
    def _():
        Wa = Wa_ref[...]
        Wb = Wb_ref[...]
        C = C_ref[...]
        mu3 = jnp.dot(Spf_ref[...], Wa + Wb, preferred_element_type=F32, precision=lax.Precision.HIGHEST) / N
        d1 = jnp.sum(Wa * jnp.dot(Mpf_ref[...], Wa, preferred_element_type=F32, precision=lax.Precision.HIGHEST), axis=0, keepdims=True)
        d2 = jnp.sum(Wa * jnp.dot(C, Wb, preferred_element_type=F32, precision=lax.Precision.HIGHEST), axis=0, keepdims=True)
        d3 = jnp.sum(Wb * jnp.dot(C, Wb, preferred_element_type=F32, precision=lax.Precision.HIGHEST), axis=0, keepdims=True)
        var3 = (d1 + 2.0 * d2 + d3) / N - mu3 * mu3
        a3 = g3_ref[...] / jnp.sqrt(var3 + 1e-5)
        aff_scr[0:1, :] = a3
        aff_scr[1:2, :] = b3_ref[...] - mu3 * a3

    t3 = jnp.dot(pf0_ref[...], Wa_ref[...], preferred_element_type=F32) + g_ref[...]
    pf1_ref[...] = jnp.maximum(t3 * aff_scr[0:1, :] + aff_scr[1:2, :], 0.0)


def _run_kE(pf0, g, Wa, Wb, C, Mpf, Spf, g3, b3):
    cmap = lambda i: (0, 0)
    return pl.pallas_call(
        _kE_body,
        grid=(_NBE,),
        in_specs=[pl.BlockSpec((_BE, 128), lambda i: (i, 0)),
                  pl.BlockSpec((_BE, 128), lambda i: (i, 0)),
                  pl.BlockSpec((128, 128), cmap),
                  pl.BlockSpec((128, 128), cmap),
                  pl.BlockSpec((128, 128), cmap),
                  pl.BlockSpec((128, 128), cmap),
                  pl.BlockSpec((1, 128), cmap),
                  pl.BlockSpec((1, 128), cmap),
                  pl.BlockSpec((1, 128), cmap)],
        out_specs=pl.BlockSpec((_BE, 128), lambda i: (i, 0)),
        out_shape=jax.ShapeDtypeStruct((N, 128), F32),
        scratch_shapes=[pltpu.VMEM((2, 128), F32)],
    )(pf0, g, Wa, Wb, C, Mpf, Spf, g3, b3)


def _kF_body(s0a_ref, s0b_ref, s1a_ref, s1b_ref, cta_ref, ctb_ref,
             Wf0_ref, Wf1_ref, bf_ref, out_ref):
    cnt = (cta_ref[...] + ctb_ref[...])[:, 0:1]
    vf0 = (s0a_ref[...] + s0b_ref[...]) / cnt
    vf1 = (s1a_ref[...] + s1b_ref[...]) / cnt
    out_ref[...] = (jnp.dot(vf0, Wf0_ref[...], preferred_element_type=F32)
                    + jnp.dot(vf1, Wf1_ref[...], preferred_element_type=F32)
                    + bf_ref[...])


def _run_kF(sums0, sums1, counts, Wf0, Wf1, bf):
    cmap = lambda i: (0, 0)
    return pl.pallas_call(
        _kF_body,
        grid=(_NBK,),
        in_specs=[pl.BlockSpec((_BK, 128), lambda i: (i, 0)),
                  pl.BlockSpec((_BK, 128), lambda i: (i + _NBK, 0)),
                  pl.BlockSpec((_BK, 128), lambda i: (i, 0)),
                  pl.BlockSpec((_BK, 128), lambda i: (i + _NBK, 0)),
                  pl.BlockSpec((_BK, 128), lambda i: (i, 0)),
                  pl.BlockSpec((_BK, 128), lambda i: (i + _NBK, 0)),
                  pl.BlockSpec((128, 128), cmap),
                  pl.BlockSpec((128, 128), cmap),
                  pl.BlockSpec((1, 128), cmap)],
        out_specs=pl.BlockSpec((_BK, 128), lambda i: (i, 0)),
        out_shape=jax.ShapeDtypeStruct((KP, 128), F32),
    )(sums0, sums0, sums1, sums1, counts, counts, Wf0, Wf1, bf)


def kernel(coors, points, features, f_cluster, W_r0, g_r0, b_r0, W_r1, g_r1,
           b_r1, W_v0, g_v0, b_v0, W_v1, g_v1, b_v1, W_f, b_f):
    coors_i32 = coors.astype(jnp.int32)
    fc = f_cluster
    feat0 = jnp.concatenate([points, features], axis=1)
    r2 = lambda v: v.reshape(1, -1)

    inv = _compute_inv(coors_i32)

    S1, M1 = _run_kA(fc)
    S2, M2 = _run_kB(fc, S1, M1, W_r0, r2(g_r0), r2(b_r0))
    S3, M3 = _run_kC(fc, feat0, S1, M1, W_r0, r2(g_r0), r2(b_r0),
                     S2, M2, W_r1, r2(g_r1), r2(b_r1))
    pf0, Spf, Mpf = _run_kD(fc, feat0, S1, M1, W_r0, r2(g_r0), r2(b_r0),
                            S2, M2, W_r1, r2(g_r1), r2(b_r1),
                            S3, M3, W_v0, r2(g_v0), r2(b_v0))

    sums0 = _seg_sums_sc(pf0, inv)
    counts = _seg_counts_sc(inv)

    Wa = W_v1[:128]
    Wb = W_v1[128:]
    u, C = _run_kM(sums0, counts, Wb)

    g = _gather_sc(u, inv)
    pf1 = _run_kE(pf0, g, Wa, Wb, C, Mpf, Spf, r2(g_v1), r2(b_v1))

    sums1 = _seg_sums_sc(pf1, inv)

    vox = _run_kF(sums0, sums1, counts, W_f[:128], W_f[128:], r2(b_f))
    return pf1, vox[:K]


# speedup vs baseline: 1.8311x; 1.0900x over previous
import functools
import jax
import jax.numpy as jnp
from jax import lax
from jax.experimental import pallas as pl
from jax.experimental.pallas import tpu as pltpu
from jax.experimental.pallas import tpu_sc as plsc

N = 320000
K = 10000
KP = 10240
F32 = jnp.float32

_BI = 2000
_NBI = N // _BI


def _inv_body(coors_ref, fc_ref, inv_ref, S1_ref, M1_ref, carry):
    i = pl.program_id(0)
    blk = coors_ref[0]

    @pl.when(i == 0)
    def _():
        carry[0] = blk[0, 0]
        carry[1] = 0
        S1_ref[...] = jnp.zeros_like(S1_ref)
        M1_ref[...] = jnp.zeros_like(M1_ref)

    prev = carry[0]
    shifted = jnp.roll(blk, 1, axis=1)
    col = lax.broadcasted_iota(jnp.int32, blk.shape, 1)
    shifted = jnp.where(col == 0, prev, shifted)
    flags = (blk != shifted).astype(jnp.int32)
    x = flags
    sh = 1
    while sh < _BI:
        x = x + jnp.where(col >= sh, jnp.roll(x, sh, axis=1), 0)
        sh *= 2
    inv_ref[0] = carry[1] + x
    carry[1] = carry[1] + jnp.sum(flags)
    carry[0] = blk[0, _BI - 1]
    fc = fc_ref[...]
    S1_ref[...] += jnp.sum(fc, axis=0, keepdims=True)
    M1_ref[...] += lax.dot_general(fc, fc, (((0,), (0,)), ((), ())),
                                   preferred_element_type=F32, precision=lax.Precision.HIGHEST)


def _compute_inv(coors_i32, fc):
    c3 = coors_i32.reshape(_NBI, 1, _BI)
    inv, S1, M1 = pl.pallas_call(
        _inv_body,
        grid=(_NBI,),
        in_specs=[pl.BlockSpec((1, 1, _BI), lambda i: (i, 0, 0)),
                  pl.BlockSpec((_BI, 3), lambda i: (i, 0))],
        out_specs=[pl.BlockSpec((1, 1, _BI), lambda i: (i, 0, 0)),
                   pl.BlockSpec((1, 3), lambda i: (0, 0)),
                   pl.BlockSpec((3, 3), lambda i: (0, 0))],
        out_shape=[jax.ShapeDtypeStruct((_NBI, 1, _BI), jnp.int32),
                   jax.ShapeDtypeStruct((1, 3), F32),
                   jax.ShapeDtypeStruct((3, 3), F32)],
        scratch_shapes=[pltpu.SMEM((2,), jnp.int32)],
    )(c3, fc)
    return inv.reshape(N), S1, M1


def _bn_affine(S, M, W, g, b):
    mu = jnp.dot(S, W, preferred_element_type=F32, precision=lax.Precision.HIGHEST) / N
    ex2 = jnp.sum(W * jnp.dot(M, W, preferred_element_type=F32, precision=lax.Precision.HIGHEST), axis=0, keepdims=True) / N
    var = ex2 - mu * mu
    a = g / jnp.sqrt(var + 1e-5)
    c = b - mu * a
    return a, c


_BA = 4000
_NBA = N // _BA


def _kB_body(fc_ref, S1_ref, M1_ref, Wr0_ref, g0_ref, b0_ref, S2_ref, M2_ref):
    i = pl.program_id(0)

    @pl.when(i == 0)
    def _():
        S2_ref[...] = jnp.zeros_like(S2_ref)
        M2_ref[...] = jnp.zeros_like(M2_ref)

    Wr0 = Wr0_ref[...]
    a0, c0 = _bn_affine(S1_ref[...], M1_ref[...], Wr0, g0_ref[...], b0_ref[...])
    fc = fc_ref[...]
    h = jnp.maximum(jnp.dot(fc, Wr0, preferred_element_type=F32) * a0 + c0, 0.0)
    S2_ref[...] += jnp.sum(h, axis=0, keepdims=True)
    M2_ref[...] += lax.dot_general(h, h, (((0,), (0,)), ((), ())),
                                   preferred_element_type=F32, precision=lax.Precision.HIGHEST)


def _run_kB(fc, S1, M1, Wr0, g0, b0):
    return pl.pallas_call(
        _kB_body,
        grid=(_NBA,),
        in_specs=[pl.BlockSpec((_BA, 3), lambda i: (i, 0)),
                  pl.BlockSpec((1, 3), lambda i: (0, 0)),
                  pl.BlockSpec((3, 3), lambda i: (0, 0)),
                  pl.BlockSpec((3, 16), lambda i: (0, 0)),
                  pl.BlockSpec((1, 16), lambda i: (0, 0)),
                  pl.BlockSpec((1, 16), lambda i: (0, 0))],
        out_specs=[pl.BlockSpec((1, 16), lambda i: (0, 0)),
                   pl.BlockSpec((16, 16), lambda i: (0, 0))],
        out_shape=[jax.ShapeDtypeStruct((1, 16), F32),
                   jax.ShapeDtypeStruct((16, 16), F32)],
    )(fc, S1, M1, Wr0, g0, b0)


def _kC_body(fc_ref, f0_ref, S1_ref, M1_ref, Wr0_ref, g0_ref, b0_ref,
             S2_ref, M2_ref, Wr1_ref, g1_ref, b1_ref, S3_ref, M3_ref):
    i = pl.program_id(0)

    @pl.when(i == 0)
    def _():
        S3_ref[...] = jnp.zeros_like(S3_ref)
        M3_ref[...] = jnp.zeros_like(M3_ref)

    Wr0 = Wr0_ref[...]
    Wr1 = Wr1_ref[...]
    a0, c0 = _bn_affine(S1_ref[...], M1_ref[...], Wr0, g0_ref[...], b0_ref[...])
    a1, c1 = _bn_affine(S2_ref[...], M2_ref[...], Wr1, g1_ref[...], b1_ref[...])
    fc = fc_ref[...]
    h = jnp.maximum(jnp.dot(fc, Wr0, preferred_element_type=F32) * a0 + c0, 0.0)
    rel = jnp.maximum(jnp.dot(h, Wr1, preferred_element_type=F32) * a1 + c1, 0.0)
    feats = f0_ref[...] * rel
    S3_ref[...] += jnp.sum(feats, axis=0, keepdims=True)
    M3_ref[...] += lax.dot_general(feats, feats, (((0,), (0,)), ((), ())),
                                   preferred_element_type=F32, precision=lax.Precision.HIGHEST)


def _run_kC(fc, f0, S1, M1, Wr0, g0, b0, S2, M2, Wr1, g1, b1):
    cmap = lambda i: (0, 0)
    return pl.pallas_call(
        _kC_body,
        grid=(_NBA,),
        in_specs=[pl.BlockSpec((_BA, 3), lambda i: (i, 0)),
                  pl.BlockSpec((_BA, 64), lambda i: (i, 0)),
                  pl.BlockSpec((1, 3), cmap), pl.BlockSpec((3, 3), cmap),
                  pl.BlockSpec((3, 16), cmap), pl.BlockSpec((1, 16), cmap),
                  pl.BlockSpec((1, 16), cmap), pl.BlockSpec((1, 16), cmap),
                  pl.BlockSpec((16, 16), cmap), pl.BlockSpec((16, 64), cmap),
                  pl.BlockSpec((1, 64), cmap), pl.BlockSpec((1, 64), cmap)],
        out_specs=[pl.BlockSpec((1, 64), cmap),
                   pl.BlockSpec((64, 64), cmap)],
        out_shape=[jax.ShapeDtypeStruct((1, 64), F32),
                   jax.ShapeDtypeStruct((64, 64), F32)],
    )(fc, f0, S1, M1, Wr0, g0, b0, S2, M2, Wr1, g1, b1)


_BD = 2000
_NBD = N // _BD


def _kD_body(fc_ref, f0_ref, S1_ref, M1_ref, Wr0_ref, g0_ref, b0_ref,
             S2_ref, M2_ref, Wr1_ref, g1_ref, b1_ref,
             S3_ref, M3_ref, Wv0_ref, g2_ref, b2_ref,
             pf0_ref, Spf_ref, Mpf_ref, aff_scr):
    i = pl.program_id(0)

    @pl.when(i == 0)
    def _():
        Spf_ref[...] = jnp.zeros_like(Spf_ref)
        Mpf_ref[...] = jnp.zeros_like(Mpf_ref)
        a2, c2 = _bn_affine(S3_ref[...], M3_ref[...], Wv0_ref[...],
                            g2_ref[...], b2_ref[...])
        aff_scr[0:1, :] = a2
        aff_scr[1:2, :] = c2

    Wr0 = Wr0_ref[...]
    Wr1 = Wr1_ref[...]
    a0, c0 = _bn_affine(S1_ref[...], M1_ref[...], Wr0, g0_ref[...], b0_ref[...])
    a1, c1 = _bn_affine(S2_ref[...], M2_ref[...], Wr1, g1_ref[...], b1_ref[...])
    fc = fc_ref[...]
    h = jnp.maximum(jnp.dot(fc, Wr0, preferred_element_type=F32) * a0 + c0, 0.0)
    rel = jnp.maximum(jnp.dot(h, Wr1, preferred_element_type=F32) * a1 + c1, 0.0)
    feats = f0_ref[...] * rel
    t2 = jnp.dot(feats, Wv0_ref[...], preferred_element_type=F32)
    pf0 = jnp.maximum(t2 * aff_scr[0:1, :] + aff_scr[1:2, :], 0.0)
    pf0_ref[...] = pf0
    Spf_ref[...] += jnp.sum(pf0, axis=0, keepdims=True)
    Mpf_ref[...] += lax.dot_general(pf0, pf0, (((0,), (0,)), ((), ())),
                                    preferred_element_type=F32, precision=lax.Precision.HIGHEST)


def _run_kD(fc, f0, S1, M1, Wr0, g0, b0, S2, M2, Wr1, g1, b1, S3, M3, Wv0, g2, b2):
    cmap = lambda i: (0, 0)
    return pl.pallas_call(
        _kD_body,
        grid=(_NBD,),
        in_specs=[pl.BlockSpec((_BD, 3), lambda i: (i, 0)),
                  pl.BlockSpec((_BD, 64), lambda i: (i, 0)),
                  pl.BlockSpec((1, 3), cmap), pl.BlockSpec((3, 3), cmap),
                  pl.BlockSpec((3, 16), cmap), pl.BlockSpec((1, 16), cmap),
                  pl.BlockSpec((1, 16), cmap), pl.BlockSpec((1, 16), cmap),
                  pl.BlockSpec((16, 16), cmap), pl.BlockSpec((16, 64), cmap),
                  pl.BlockSpec((1, 64), cmap), pl.BlockSpec((1, 64), cmap),
                  pl.BlockSpec((1, 64), cmap), pl.BlockSpec((64, 64), cmap),
                  pl.BlockSpec((64, 128), cmap), pl.BlockSpec((1, 128), cmap),
                  pl.BlockSpec((1, 128), cmap)],
        out_specs=[pl.BlockSpec((_BD, 128), lambda i: (i, 0)),
                   pl.BlockSpec((1, 128), cmap),
                   pl.BlockSpec((128, 128), cmap)],
        out_shape=[jax.ShapeDtypeStruct((N, 128), F32),
                   jax.ShapeDtypeStruct((1, 128), F32),
                   jax.ShapeDtypeStruct((128, 128), F32)],
        scratch_shapes=[pltpu.VMEM((2, 128), F32)],
    )(fc, f0, S1, M1, Wr0, g0, b0, S2, M2, Wr1, g1, b1, S3, M3, Wv0, g2, b2)


_RW = N // 32
_CH = 80
_NCH = _RW // _CH
_ZR = KP // 16


def _sc_sums_body(pf_hbm, inv_hbm, zer128, sums_hbm, idx_v, rows_v, accum):
    cid = lax.axis_index("c")
    sid = lax.axis_index("s")
    wid = sid * 2 + cid

    pltpu.sync_copy(zer128, accum.at[pl.ds(sid * _ZR, _ZR)])
    plsc.subcore_barrier()

    def body(t, carry):
        base = wid * _RW + t * _CH
        pltpu.sync_copy(inv_hbm.at[pl.ds(base, _CH)], idx_v)
        pltpu.sync_copy(pf_hbm.at[pl.ds(base, _CH)], rows_v)
        pltpu.sync_copy(rows_v, accum.at[idx_v], add=True)
        return carry

    lax.fori_loop(0, _NCH, body, 0)
    plsc.subcore_barrier()

    pltpu.sync_copy(accum.at[pl.ds(sid * _ZR, _ZR)],
                    sums_hbm.at[pl.ds(cid * KP + sid * _ZR, _ZR)])


def _seg_sums_sc(pf, inv_i32):
    mesh = plsc.VectorSubcoreMesh(core_axis_name="c", subcore_axis_name="s")
    run = pl.kernel(
        _sc_sums_body,
        mesh=mesh,
        out_type=[jax.ShapeDtypeStruct((2 * KP, 128), F32)],
        scratch_types=[pltpu.VMEM((_CH,), jnp.int32),
                       pltpu.VMEM((_CH, 128), F32),
                       pltpu.VMEM_SHARED((KP, 128), F32)],
    )
    (sums,) = run(pf, inv_i32, jnp.zeros((_ZR, 128), F32))
    return sums


def _sc_counts_body(inv_hbm, zer128, ones128, counts_hbm, idx_v, ones_v, accum):
    cid = lax.axis_index("c")
    sid = lax.axis_index("s")
    wid = sid * 2 + cid

    pltpu.sync_copy(zer128, accum.at[pl.ds(sid * _ZR, _ZR)])
    pltpu.sync_copy(ones128, ones_v)
    plsc.subcore_barrier()

    def body(t, carry):
        base = wid * _RW + t * _CH
        pltpu.sync_copy(inv_hbm.at[pl.ds(base, _CH)], idx_v)
        pltpu.sync_copy(ones_v, accum.at[idx_v], add=True)
        return carry

    lax.fori_loop(0, _NCH, body, 0)
    plsc.subcore_barrier()

    pltpu.sync_copy(accum.at[pl.ds(sid * _ZR, _ZR)],
                    counts_hbm.at[pl.ds(cid * KP + sid * _ZR, _ZR)])


def _sc_gather_body(u_hbm, inv_hbm, g_hbm, idx_a, idx_b, rows_a, rows_b,
                    sem_a, sem_b):
    cid = lax.axis_index("c")
    sid = lax.axis_index("s")
    wid = sid * 2 + cid
    base0 = wid * _RW

    def pair(p, carry):
        ba = base0 + (2 * p) * _CH
        bb = base0 + (2 * p + 1) * _CH
        pltpu.sync_copy(inv_hbm.at[pl.ds(ba, _CH)], idx_a)
        ga = pltpu.async_copy(u_hbm.at[idx_a], rows_a, sem_a)
        pltpu.sync_copy(inv_hbm.at[pl.ds(bb, _CH)], idx_b)
        gb = pltpu.async_copy(u_hbm.at[idx_b], rows_b, sem_b)
        ga.wait()
        pltpu.sync_copy(rows_a, g_hbm.at[pl.ds(ba, _CH)])
        gb.wait()
        pltpu.sync_copy(rows_b, g_hbm.at[pl.ds(bb, _CH)])
        return carry

    lax.fori_loop(0, _NCH // 2, pair, 0)
    bt = base0 + (_NCH - 1) * _CH
    pltpu.sync_copy(inv_hbm.at[pl.ds(bt, _CH)], idx_a)
    pltpu.async_copy(u_hbm.at[idx_a], rows_a, sem_a).wait()
    pltpu.sync_copy(rows_a, g_hbm.at[pl.ds(bt, _CH)])


def _gather_sc(u, inv_i32):
    mesh = plsc.VectorSubcoreMesh(core_axis_name="c", subcore_axis_name="s")
    run = pl.kernel(
        _sc_gather_body,
        mesh=mesh,
        out_type=[jax.ShapeDtypeStruct((N, 128), F32)],
        scratch_types=[pltpu.VMEM((_CH,), jnp.int32),
                       pltpu.VMEM((_CH,), jnp.int32),
                       pltpu.VMEM((_CH, 128), F32),
                       pltpu.VMEM((_CH, 128), F32),
                       pltpu.SemaphoreType.DMA,
                       pltpu.SemaphoreType.DMA],
    )
    (g,) = run(u, inv_i32)
    return g


def _seg_counts_sc(inv_i32):
    mesh = plsc.VectorSubcoreMesh(core_axis_name="c", subcore_axis_name="s")
    run = pl.kernel(
        _sc_counts_body,
        mesh=mesh,
        out_type=[jax.ShapeDtypeStruct((2 * KP, 128), F32)],
        scratch_types=[pltpu.VMEM((_CH,), jnp.int32),
                       pltpu.VMEM((_CH, 128), F32),
                       pltpu.VMEM_SHARED((KP, 128), F32)],
    )
    (counts,) = run(inv_i32, jnp.zeros((_ZR, 128), F32),
                    jnp.ones((_CH, 128), F32))
    return counts


_BK = 512
_NBK = KP // _BK


def _kM_body(s0a_ref, s0b_ref, cta_ref, ctb_ref, Wb_ref, u_ref, C_ref):
    i = pl.program_id(0)

    @pl.when(i == 0)
    def _():
        C_ref[...] = jnp.zeros_like(C_ref)

    s0 = s0a_ref[...] + s0b_ref[...]
    cnt = (cta_ref[...] + ctb_ref[...])[:, 0:1]
    vf0 = jnp.where(cnt > 0, s0 / cnt, 0.0)
    u_ref[...] = jnp.dot(vf0, Wb_ref[...], preferred_element_type=F32)
    C_ref[...] += lax.dot_general(s0, vf0, (((0,), (0,)), ((), ())),
                                  preferred_element_type=F32, precision=lax.Precision.HIGHEST)


def _run_kM(sums, counts, Wb):
    cmap = lambda i: (0, 0)
    return pl.pallas_call(
        _kM_body,
        grid=(_NBK,),
        in_specs=[pl.BlockSpec((_BK, 128), lambda i: (i, 0)),
                  pl.BlockSpec((_BK, 128), lambda i: (i + _NBK, 0)),
                  pl.BlockSpec((_BK, 128), lambda i: (i, 0)),
                  pl.BlockSpec((_BK, 128), lambda i: (i + _NBK, 0)),
                  pl.BlockSpec((128, 128), cmap)],
        out_specs=[pl.BlockSpec((_BK, 128), lambda i: (i, 0)),
                   pl.BlockSpec((128, 128), cmap)],
        out_shape=[jax.ShapeDtypeStruct((KP, 128), F32),
                   jax.ShapeDtypeStruct((128, 128), F32)],
    )(sums, sums, counts, counts, Wb)


_BE = 512
_NBE = N // _BE


def _kE_body(pf0_ref, g_ref, Wa_ref, Wb_ref, C_ref, Mpf_ref,
             Spf_ref, g3_ref, b3_ref, pf1_ref, aff_scr):
    i = pl.program_id(0)

    @pl.when(i == 0)
    def _():
        Wa = Wa_ref[...]
        Wb = Wb_ref[...]
        C = C_ref[...]
        mu3 = jnp.dot(Spf_ref[...], Wa + Wb, preferred_element_type=F32, precision=lax.Precision.HIGHEST) / N
        d1 = jnp.sum(Wa * jnp.dot(Mpf_ref[...], Wa, preferred_element_type=F32, precision=lax.Precision.HIGHEST), axis=0, keepdims=True)
        d2 = jnp.sum(Wa * jnp.dot(C, Wb, preferred_element_type=F32, precision=lax.Precision.HIGHEST), axis=0, keepdims=True)
        d3 = jnp.sum(Wb * jnp.dot(C, Wb, preferred_element_type=F32, precision=lax.Precision.HIGHEST), axis=0, keepdims=True)
        var3 = (d1 + 2.0 * d2 + d3) / N - mu3 * mu3
        a3 = g3_ref[...] / jnp.sqrt(var3 + 1e-5)
        aff_scr[0:1, :] = a3
        aff_scr[1:2, :] = b3_ref[...] - mu3 * a3

    t3 = jnp.dot(pf0_ref[...], Wa_ref[...], preferred_element_type=F32) + g_ref[...]
    pf1_ref[...] = jnp.maximum(t3 * aff_scr[0:1, :] + aff_scr[1:2, :], 0.0)


def _run_kE(pf0, g, Wa, Wb, C, Mpf, Spf, g3, b3):
    cmap = lambda i: (0, 0)
    return pl.pallas_call(
        _kE_body,
        grid=(_NBE,),
        in_specs=[pl.BlockSpec((_BE, 128), lambda i: (i, 0)),
                  pl.BlockSpec((_BE, 128), lambda i: (i, 0)),
                  pl.BlockSpec((128, 128), cmap),
                  pl.BlockSpec((128, 128), cmap),
                  pl.BlockSpec((128, 128), cmap),
                  pl.BlockSpec((128, 128), cmap),
                  pl.BlockSpec((1, 128), cmap),
                  pl.BlockSpec((1, 128), cmap),
                  pl.BlockSpec((1, 128), cmap)],
        out_specs=pl.BlockSpec((_BE, 128), lambda i: (i, 0)),
        out_shape=jax.ShapeDtypeStruct((N, 128), F32),
        scratch_shapes=[pltpu.VMEM((2, 128), F32)],
    )(pf0, g, Wa, Wb, C, Mpf, Spf, g3, b3)


def _kF_body(s0a_ref, s0b_ref, s1a_ref, s1b_ref, cta_ref, ctb_ref,
             Wf0_ref, Wf1_ref, bf_ref, out_ref):
    cnt = (cta_ref[...] + ctb_ref[...])[:, 0:1]
    vf0 = (s0a_ref[...] + s0b_ref[...]) / cnt
    vf1 = (s1a_ref[...] + s1b_ref[...]) / cnt
    out_ref[...] = (jnp.dot(vf0, Wf0_ref[...], preferred_element_type=F32)
                    + jnp.dot(vf1, Wf1_ref[...], preferred_element_type=F32)
                    + bf_ref[...])


def _run_kF(sums0, sums1, counts, Wf0, Wf1, bf):
    cmap = lambda i: (0, 0)
    return pl.pallas_call(
        _kF_body,
        grid=(_NBK,),
        in_specs=[pl.BlockSpec((_BK, 128), lambda i: (i, 0)),
                  pl.BlockSpec((_BK, 128), lambda i: (i + _NBK, 0)),
                  pl.BlockSpec((_BK, 128), lambda i: (i, 0)),
                  pl.BlockSpec((_BK, 128), lambda i: (i + _NBK, 0)),
                  pl.BlockSpec((_BK, 128), lambda i: (i, 0)),
                  pl.BlockSpec((_BK, 128), lambda i: (i + _NBK, 0)),
                  pl.BlockSpec((128, 128), cmap),
                  pl.BlockSpec((128, 128), cmap),
                  pl.BlockSpec((1, 128), cmap)],
        out_specs=pl.BlockSpec((_BK, 128), lambda i: (i, 0)),
        out_shape=jax.ShapeDtypeStruct((KP, 128), F32),
    )(sums0, sums0, sums1, sums1, counts, counts, Wf0, Wf1, bf)


def kernel(coors, points, features, f_cluster, W_r0, g_r0, b_r0, W_r1, g_r1,
           b_r1, W_v0, g_v0, b_v0, W_v1, g_v1, b_v1, W_f, b_f):
    coors_i32 = coors.astype(jnp.int32)
    fc = f_cluster
    feat0 = jnp.concatenate([points, features], axis=1)
    r2 = lambda v: v.reshape(1, -1)

    inv, S1, M1 = _compute_inv(coors_i32, fc)
    S2, M2 = _run_kB(fc, S1, M1, W_r0, r2(g_r0), r2(b_r0))
    S3, M3 = _run_kC(fc, feat0, S1, M1, W_r0, r2(g_r0), r2(b_r0),
                     S2, M2, W_r1, r2(g_r1), r2(b_r1))
    pf0, Spf, Mpf = _run_kD(fc, feat0, S1, M1, W_r0, r2(g_r0), r2(b_r0),
                            S2, M2, W_r1, r2(g_r1), r2(b_r1),
                            S3, M3, W_v0, r2(g_v0), r2(b_v0))

    sums0 = _seg_sums_sc(pf0, inv)
    counts = _seg_counts_sc(inv)

    Wa = W_v1[:128]
    Wb = W_v1[128:]
    u, C = _run_kM(sums0, counts, Wb)

    g = _gather_sc(u, inv)
    pf1 = _run_kE(pf0, g, Wa, Wb, C, Mpf, Spf, r2(g_v1), r2(b_v1))

    sums1 = _seg_sums_sc(pf1, inv)

    vox = _run_kF(sums0, sums1, counts, W_f[:128], W_f[128:], r2(b_f))
    return pf1, vox[:K]


# speedup vs baseline: 1.9997x; 1.0921x over previous
import functools
import jax
import jax.numpy as jnp
from jax import lax
from jax.experimental import pallas as pl
from jax.experimental.pallas import tpu as pltpu
from jax.experimental.pallas import tpu_sc as plsc

N = 320000
K = 10000
KP = 10240
F32 = jnp.float32

_BI = 2000
_NBI = N // _BI


def _inv_body(coors_ref, fc_ref, inv_ref, S1_ref, M1_ref, carry):
    i = pl.program_id(0)
    blk = coors_ref[0]

    @pl.when(i == 0)
    def _():
        carry[0] = blk[0, 0]
        carry[1] = 0
        S1_ref[...] = jnp.zeros_like(S1_ref)
        M1_ref[...] = jnp.zeros_like(M1_ref)

    prev = carry[0]
    shifted = jnp.roll(blk, 1, axis=1)
    col = lax.broadcasted_iota(jnp.int32, blk.shape, 1)
    shifted = jnp.where(col == 0, prev, shifted)
    flags = (blk != shifted).astype(jnp.int32)
    x = flags
    sh = 1
    while sh < _BI:
        x = x + jnp.where(col >= sh, jnp.roll(x, sh, axis=1), 0)
        sh *= 2
    inv_ref[0] = carry[1] + x
    carry[1] = carry[1] + jnp.sum(flags)
    carry[0] = blk[0, _BI - 1]
    fc = fc_ref[...]
    S1_ref[...] += jnp.sum(fc, axis=0, keepdims=True)
    M1_ref[...] += lax.dot_general(fc, fc, (((0,), (0,)), ((), ())),
                                   preferred_element_type=F32, precision=lax.Precision.HIGHEST)


def _compute_inv(coors_i32, fc):
    c3 = coors_i32.reshape(_NBI, 1, _BI)
    inv, S1, M1 = pl.pallas_call(
        _inv_body,
        grid=(_NBI,),
        in_specs=[pl.BlockSpec((1, 1, _BI), lambda i: (i, 0, 0)),
                  pl.BlockSpec((_BI, 3), lambda i: (i, 0))],
        out_specs=[pl.BlockSpec((1, 1, _BI), lambda i: (i, 0, 0)),
                   pl.BlockSpec((1, 3), lambda i: (0, 0)),
                   pl.BlockSpec((3, 3), lambda i: (0, 0))],
        out_shape=[jax.ShapeDtypeStruct((_NBI, 1, _BI), jnp.int32),
                   jax.ShapeDtypeStruct((1, 3), F32),
                   jax.ShapeDtypeStruct((3, 3), F32)],
        scratch_shapes=[pltpu.SMEM((2,), jnp.int32)],
    )(c3, fc)
    return inv.reshape(N), S1, M1


def _bn_affine(S, M, W, g, b):
    mu = jnp.dot(S, W, preferred_element_type=F32, precision=lax.Precision.HIGHEST) / N
    ex2 = jnp.sum(W * jnp.dot(M, W, preferred_element_type=F32, precision=lax.Precision.HIGHEST), axis=0, keepdims=True) / N
    var = ex2 - mu * mu
    a = g / jnp.sqrt(var + 1e-5)
    c = b - mu * a
    return a, c


_BA = 4000
_NBA = N // _BA


def _kB_body(fc_ref, S1_ref, M1_ref, Wr0_ref, g0_ref, b0_ref, S2_ref, M2_ref):
    i = pl.program_id(0)

    @pl.when(i == 0)
    def _():
        S2_ref[...] = jnp.zeros_like(S2_ref)
        M2_ref[...] = jnp.zeros_like(M2_ref)

    Wr0 = Wr0_ref[...]
    a0, c0 = _bn_affine(S1_ref[...], M1_ref[...], Wr0, g0_ref[...], b0_ref[...])
    fc = fc_ref[...]
    h = jnp.maximum(jnp.dot(fc, Wr0, preferred_element_type=F32) * a0 + c0, 0.0)
    S2_ref[...] += jnp.sum(h, axis=0, keepdims=True)
    M2_ref[...] += lax.dot_general(h, h, (((0,), (0,)), ((), ())),
                                   preferred_element_type=F32, precision=lax.Precision.HIGHEST)


def _run_kB(fc, S1, M1, Wr0, g0, b0):
    return pl.pallas_call(
        _kB_body,
        grid=(_NBA,),
        in_specs=[pl.BlockSpec((_BA, 3), lambda i: (i, 0)),
                  pl.BlockSpec((1, 3), lambda i: (0, 0)),
                  pl.BlockSpec((3, 3), lambda i: (0, 0)),
                  pl.BlockSpec((3, 16), lambda i: (0, 0)),
                  pl.BlockSpec((1, 16), lambda i: (0, 0)),
                  pl.BlockSpec((1, 16), lambda i: (0, 0))],
        out_specs=[pl.BlockSpec((1, 16), lambda i: (0, 0)),
                   pl.BlockSpec((16, 16), lambda i: (0, 0))],
        out_shape=[jax.ShapeDtypeStruct((1, 16), F32),
                   jax.ShapeDtypeStruct((16, 16), F32)],
    )(fc, S1, M1, Wr0, g0, b0)


def _kC_body(fc_ref, f0_ref, S1_ref, M1_ref, Wr0_ref, g0_ref, b0_ref,
             S2_ref, M2_ref, Wr1_ref, g1_ref, b1_ref, S3_ref, M3_ref):
    i = pl.program_id(0)

    @pl.when(i == 0)
    def _():
        S3_ref[...] = jnp.zeros_like(S3_ref)
        M3_ref[...] = jnp.zeros_like(M3_ref)

    Wr0 = Wr0_ref[...]
    Wr1 = Wr1_ref[...]
    a0, c0 = _bn_affine(S1_ref[...], M1_ref[...], Wr0, g0_ref[...], b0_ref[...])
    a1, c1 = _bn_affine(S2_ref[...], M2_ref[...], Wr1, g1_ref[...], b1_ref[...])
    fc = fc_ref[...]
    h = jnp.maximum(jnp.dot(fc, Wr0, preferred_element_type=F32) * a0 + c0, 0.0)
    rel = jnp.maximum(jnp.dot(h, Wr1, preferred_element_type=F32) * a1 + c1, 0.0)
    feats = f0_ref[...] * rel
    S3_ref[...] += jnp.sum(feats, axis=0, keepdims=True)
    M3_ref[...] += lax.dot_general(feats, feats, (((0,), (0,)), ((), ())),
                                   preferred_element_type=F32, precision=lax.Precision.HIGHEST)


def _run_kC(fc, f0, S1, M1, Wr0, g0, b0, S2, M2, Wr1, g1, b1):
    cmap = lambda i: (0, 0)
    return pl.pallas_call(
        _kC_body,
        grid=(_NBA,),
        in_specs=[pl.BlockSpec((_BA, 3), lambda i: (i, 0)),
                  pl.BlockSpec((_BA, 64), lambda i: (i, 0)),
                  pl.BlockSpec((1, 3), cmap), pl.BlockSpec((3, 3), cmap),
                  pl.BlockSpec((3, 16), cmap), pl.BlockSpec((1, 16), cmap),
                  pl.BlockSpec((1, 16), cmap), pl.BlockSpec((1, 16), cmap),
                  pl.BlockSpec((16, 16), cmap), pl.BlockSpec((16, 64), cmap),
                  pl.BlockSpec((1, 64), cmap), pl.BlockSpec((1, 64), cmap)],
        out_specs=[pl.BlockSpec((1, 64), cmap),
                   pl.BlockSpec((64, 64), cmap)],
        out_shape=[jax.ShapeDtypeStruct((1, 64), F32),
                   jax.ShapeDtypeStruct((64, 64), F32)],
    )(fc, f0, S1, M1, Wr0, g0, b0, S2, M2, Wr1, g1, b1)


_BD = 2000
_NBD = N // _BD


def _kD_body(fc_ref, f0_ref, S1_ref, M1_ref, Wr0_ref, g0_ref, b0_ref,
             S2_ref, M2_ref, Wr1_ref, g1_ref, b1_ref,
             S3_ref, M3_ref, Wv0_ref, g2_ref, b2_ref,
             pf0_ref, Spf_ref, Mpf_ref, aff_scr):
    i = pl.program_id(0)

    @pl.when(i == 0)
    def _():
        Spf_ref[...] = jnp.zeros_like(Spf_ref)
        Mpf_ref[...] = jnp.zeros_like(Mpf_ref)
        a2, c2 = _bn_affine(S3_ref[...], M3_ref[...], Wv0_ref[...],
                            g2_ref[...], b2_ref[...])
        aff_scr[0:1, :] = a2
        aff_scr[1:2, :] = c2

    Wr0 = Wr0_ref[...]
    Wr1 = Wr1_ref[...]
    a0, c0 = _bn_affine(S1_ref[...], M1_ref[...], Wr0, g0_ref[...], b0_ref[...])
    a1, c1 = _bn_affine(S2_ref[...], M2_ref[...], Wr1, g1_ref[...], b1_ref[...])
    fc = fc_ref[...]
    h = jnp.maximum(jnp.dot(fc, Wr0, preferred_element_type=F32) * a0 + c0, 0.0)
    rel = jnp.maximum(jnp.dot(h, Wr1, preferred_element_type=F32) * a1 + c1, 0.0)
    feats = f0_ref[...] * rel
    t2 = jnp.dot(feats, Wv0_ref[...], preferred_element_type=F32)
    pf0 = jnp.maximum(t2 * aff_scr[0:1, :] + aff_scr[1:2, :], 0.0)
    pf0_ref[...] = pf0
    Spf_ref[...] += jnp.sum(pf0, axis=0, keepdims=True)
    Mpf_ref[...] += lax.dot_general(pf0, pf0, (((0,), (0,)), ((), ())),
                                    preferred_element_type=F32, precision=lax.Precision.HIGHEST)


def _run_kD(fc, f0, S1, M1, Wr0, g0, b0, S2, M2, Wr1, g1, b1, S3, M3, Wv0, g2, b2):
    cmap = lambda i: (0, 0)
    return pl.pallas_call(
        _kD_body,
        grid=(_NBD,),
        in_specs=[pl.BlockSpec((_BD, 3), lambda i: (i, 0)),
                  pl.BlockSpec((_BD, 64), lambda i: (i, 0)),
                  pl.BlockSpec((1, 3), cmap), pl.BlockSpec((3, 3), cmap),
                  pl.BlockSpec((3, 16), cmap), pl.BlockSpec((1, 16), cmap),
                  pl.BlockSpec((1, 16), cmap), pl.BlockSpec((1, 16), cmap),
                  pl.BlockSpec((16, 16), cmap), pl.BlockSpec((16, 64), cmap),
                  pl.BlockSpec((1, 64), cmap), pl.BlockSpec((1, 64), cmap),
                  pl.BlockSpec((1, 64), cmap), pl.BlockSpec((64, 64), cmap),
                  pl.BlockSpec((64, 128), cmap), pl.BlockSpec((1, 128), cmap),
                  pl.BlockSpec((1, 128), cmap)],
        out_specs=[pl.BlockSpec((_BD, 128), lambda i: (i, 0)),
                   pl.BlockSpec((1, 128), cmap),
                   pl.BlockSpec((128, 128), cmap)],
        out_shape=[jax.ShapeDtypeStruct((N, 128), F32),
                   jax.ShapeDtypeStruct((1, 128), F32),
                   jax.ShapeDtypeStruct((128, 128), F32)],
        scratch_shapes=[pltpu.VMEM((2, 128), F32)],
    )(fc, f0, S1, M1, Wr0, g0, b0, S2, M2, Wr1, g1, b1, S3, M3, Wv0, g2, b2)


_RW = N // 32
_CH = 80
_NCH = _RW // _CH
_ZR = KP // 16


def _sc_sums_body(pf_hbm, inv_hbm, zer128, sums_hbm, idx_a, idx_b,
                  rows_a, rows_b, sem_a, sem_b, accum):
    cid = lax.axis_index("c")
    sid = lax.axis_index("s")
    wid = sid * 2 + cid
    base0 = wid * _RW

    pltpu.sync_copy(zer128, accum.at[pl.ds(sid * _ZR, _ZR)])
    plsc.subcore_barrier()

    def pair(p, carry):
        ba = base0 + (2 * p) * _CH
        bb = base0 + (2 * p + 1) * _CH
        la = pltpu.async_copy(pf_hbm.at[pl.ds(ba, _CH)], rows_a, sem_a)
        pltpu.sync_copy(inv_hbm.at[pl.ds(ba, _CH)], idx_a)
        lb = pltpu.async_copy(pf_hbm.at[pl.ds(bb, _CH)], rows_b, sem_b)
        pltpu.sync_copy(inv_hbm.at[pl.ds(bb, _CH)], idx_b)
        la.wait()
        pltpu.sync_copy(rows_a, accum.at[idx_a], add=True)
        lb.wait()
        pltpu.sync_copy(rows_b, accum.at[idx_b], add=True)
        return carry

    lax.fori_loop(0, _NCH // 2, pair, 0)
    bt = base0 + (_NCH - 1) * _CH
    pltpu.sync_copy(inv_hbm.at[pl.ds(bt, _CH)], idx_a)
    pltpu.sync_copy(pf_hbm.at[pl.ds(bt, _CH)], rows_a)
    pltpu.sync_copy(rows_a, accum.at[idx_a], add=True)
    plsc.subcore_barrier()

    pltpu.sync_copy(accum.at[pl.ds(sid * _ZR, _ZR)],
                    sums_hbm.at[pl.ds(cid * KP + sid * _ZR, _ZR)])


def _seg_sums_sc(pf, inv_i32):
    mesh = plsc.VectorSubcoreMesh(core_axis_name="c", subcore_axis_name="s")
    run = pl.kernel(
        _sc_sums_body,
        mesh=mesh,
        out_type=[jax.ShapeDtypeStruct((2 * KP, 128), F32)],
        scratch_types=[pltpu.VMEM((_CH,), jnp.int32),
                       pltpu.VMEM((_CH,), jnp.int32),
                       pltpu.VMEM((_CH, 128), F32),
                       pltpu.VMEM((_CH, 128), F32),
                       pltpu.SemaphoreType.DMA,
                       pltpu.SemaphoreType.DMA,
                       pltpu.VMEM_SHARED((KP, 128), F32)],
    )
    (sums,) = run(pf, inv_i32, jnp.zeros((_ZR, 128), F32))
    return sums


def _sc_counts_body(inv_hbm, zer128, ones128, counts_hbm, idx_v, ones_v, accum):
    cid = lax.axis_index("c")
    sid = lax.axis_index("s")
    wid = sid * 2 + cid

    pltpu.sync_copy(zer128, accum.at[pl.ds(sid * _ZR, _ZR)])
    pltpu.sync_copy(ones128, ones_v)
    plsc.subcore_barrier()

    def body(t, carry):
        base = wid * _RW + t * _CH
        pltpu.sync_copy(inv_hbm.at[pl.ds(base, _CH)], idx_v)
        pltpu.sync_copy(ones_v, accum.at[idx_v], add=True)
        return carry

    lax.fori_loop(0, _NCH, body, 0)
    plsc.subcore_barrier()

    pltpu.sync_copy(accum.at[pl.ds(sid * _ZR, _ZR)],
                    counts_hbm.at[pl.ds(cid * KP + sid * _ZR, _ZR)])


def _sc_gather_body(u_hbm, inv_hbm, g_hbm, idx_a, idx_b, rows_a, rows_b,
                    sem_a, sem_b):
    cid = lax.axis_index("c")
    sid = lax.axis_index("s")
    wid = sid * 2 + cid
    base0 = wid * _RW

    def pair(p, carry):
        ba = base0 + (2 * p) * _CH
        bb = base0 + (2 * p + 1) * _CH
        pltpu.sync_copy(inv_hbm.at[pl.ds(ba, _CH)], idx_a)
        ga = pltpu.async_copy(u_hbm.at[idx_a], rows_a, sem_a)
        pltpu.sync_copy(inv_hbm.at[pl.ds(bb, _CH)], idx_b)
        gb = pltpu.async_copy(u_hbm.at[idx_b], rows_b, sem_b)
        ga.wait()
        pltpu.sync_copy(rows_a, g_hbm.at[pl.ds(ba, _CH)])
        gb.wait()
        pltpu.sync_copy(rows_b, g_hbm.at[pl.ds(bb, _CH)])
        return carry

    lax.fori_loop(0, _NCH // 2, pair, 0)
    bt = base0 + (_NCH - 1) * _CH
    pltpu.sync_copy(inv_hbm.at[pl.ds(bt, _CH)], idx_a)
    pltpu.async_copy(u_hbm.at[idx_a], rows_a, sem_a).wait()
    pltpu.sync_copy(rows_a, g_hbm.at[pl.ds(bt, _CH)])


def _gather_sc(u, inv_i32):
    mesh = plsc.VectorSubcoreMesh(core_axis_name="c", subcore_axis_name="s")
    run = pl.kernel(
        _sc_gather_body,
        mesh=mesh,
        out_type=[jax.ShapeDtypeStruct((N, 128), F32)],
        scratch_types=[pltpu.VMEM((_CH,), jnp.int32),
                       pltpu.VMEM((_CH,), jnp.int32),
                       pltpu.VMEM((_CH, 128), F32),
                       pltpu.VMEM((_CH, 128), F32),
                       pltpu.SemaphoreType.DMA,
                       pltpu.SemaphoreType.DMA],
    )
    (g,) = run(u, inv_i32)
    return g


def _seg_counts_sc(inv_i32):
    mesh = plsc.VectorSubcoreMesh(core_axis_name="c", subcore_axis_name="s")
    run = pl.kernel(
        _sc_counts_body,
        mesh=mesh,
        out_type=[jax.ShapeDtypeStruct((2 * KP, 128), F32)],
        scratch_types=[pltpu.VMEM((_CH,), jnp.int32),
                       pltpu.VMEM((_CH, 128), F32),
                       pltpu.VMEM_SHARED((KP, 128), F32)],
    )
    (counts,) = run(inv_i32, jnp.zeros((_ZR, 128), F32),
                    jnp.ones((_CH, 128), F32))
    return counts


_BK = 512
_NBK = KP // _BK


def _kM_body(s0a_ref, s0b_ref, cta_ref, ctb_ref, Wb_ref, u_ref, C_ref):
    i = pl.program_id(0)

    @pl.when(i == 0)
    def _():
        C_ref[...] = jnp.zeros_like(C_ref)

    s0 = s0a_ref[...] + s0b_ref[...]
    cnt = (cta_ref[...] + ctb_ref[...])[:, 0:1]
    vf0 = jnp.where(cnt > 0, s0 / cnt, 0.0)
    u_ref[...] = jnp.dot(vf0, Wb_ref[...], preferred_element_type=F32)
    C_ref[...] += lax.dot_general(s0, vf0, (((0,), (0,)), ((), ())),
                                  preferred_element_type=F32, precision=lax.Precision.HIGHEST)


def _run_kM(sums, counts, Wb):
    cmap = lambda i: (0, 0)
    return pl.pallas_call(
        _kM_body,
        grid=(_NBK,),
        in_specs=[pl.BlockSpec((_BK, 128), lambda i: (i, 0)),
                  pl.BlockSpec((_BK, 128), lambda i: (i + _NBK, 0)),
                  pl.BlockSpec((_BK, 128), lambda i: (i, 0)),
                  pl.BlockSpec((_BK, 128), lambda i: (i + _NBK, 0)),
                  pl.BlockSpec((128, 128), cmap)],
        out_specs=[pl.BlockSpec((_BK, 128), lambda i: (i, 0)),
                   pl.BlockSpec((128, 128), cmap)],
        out_shape=[jax.ShapeDtypeStruct((KP, 128), F32),
                   jax.ShapeDtypeStruct((128, 128), F32)],
    )(sums, sums, counts, counts, Wb)


_BE = 512
_NBE = N // _BE


def _kE_body(pf0_ref, g_ref, Wa_ref, Wb_ref, C_ref, Mpf_ref,
             Spf_ref, g3_ref, b3_ref, pf1_ref, aff_scr):
    i = pl.program_id(0)

    @pl.when(i == 0)
    def _():
        Wa = Wa_ref[...]
        Wb = Wb_ref[...]
        C = C_ref[...]
        mu3 = jnp.dot(Spf_ref[...], Wa + Wb, preferred_element_type=F32, precision=lax.Precision.HIGHEST) / N
        d1 = jnp.sum(Wa * jnp.dot(Mpf_ref[...], Wa, preferred_element_type=F32, precision=lax.Precision.HIGHEST), axis=0, keepdims=True)
        d2 = jnp.sum(Wa * jnp.dot(C, Wb, preferred_element_type=F32, precision=lax.Precision.HIGHEST), axis=0, keepdims=True)
        d3 = jnp.sum(Wb * jnp.dot(C, Wb, preferred_element_type=F32, precision=lax.Precision.HIGHEST), axis=0, keepdims=True)
        var3 = (d1 + 2.0 * d2 + d3) / N - mu3 * mu3
        a3 = g3_ref[...] / jnp.sqrt(var3 + 1e-5)
        aff_scr[0:1, :] = a3
        aff_scr[1:2, :] = b3_ref[...] - mu3 * a3

    t3 = jnp.dot(pf0_ref[...], Wa_ref[...], preferred_element_type=F32) + g_ref[...]
    pf1_ref[...] = jnp.maximum(t3 * aff_scr[0:1, :] + aff_scr[1:2, :], 0.0)


def _run_kE(pf0, g, Wa, Wb, C, Mpf, Spf, g3, b3):
    cmap = lambda i: (0, 0)
    return pl.pallas_call(
        _kE_body,
        grid=(_NBE,),
        in_specs=[pl.BlockSpec((_BE, 128), lambda i: (i, 0)),
                  pl.BlockSpec((_BE, 128), lambda i: (i, 0)),
                  pl.BlockSpec((128, 128), cmap),
                  pl.BlockSpec((128, 128), cmap),
                  pl.BlockSpec((128, 128), cmap),
                  pl.BlockSpec((128, 128), cmap),
                  pl.BlockSpec((1, 128), cmap),
                  pl.BlockSpec((1, 128), cmap),
                  pl.BlockSpec((1, 128), cmap)],
        out_specs=pl.BlockSpec((_BE, 128), lambda i: (i, 0)),
        out_shape=jax.ShapeDtypeStruct((N, 128), F32),
        scratch_shapes=[pltpu.VMEM((2, 128), F32)],
    )(pf0, g, Wa, Wb, C, Mpf, Spf, g3, b3)


def _kF_body(s0a_ref, s0b_ref, s1a_ref, s1b_ref, cta_ref, ctb_ref,
             Wf0_ref, Wf1_ref, bf_ref, out_ref):
    cnt = (cta_ref[...] + ctb_ref[...])[:, 0:1]
    vf0 = (s0a_ref[...] + s0b_ref[...]) / cnt
    vf1 = (s1a_ref[...] + s1b_ref[...]) / cnt
    out_ref[...] = (jnp.dot(vf0, Wf0_ref[...], preferred_element_type=F32)
                    + jnp.dot(vf1, Wf1_ref[...], preferred_element_type=F32)
                    + bf_ref[...])


def _run_kF(sums0, sums1, counts, Wf0, Wf1, bf):
    cmap = lambda i: (0, 0)
    return pl.pallas_call(
        _kF_body,
        grid=(_NBK,),
        in_specs=[pl.BlockSpec((_BK, 128), lambda i: (i, 0)),
                  pl.BlockSpec((_BK, 128), lambda i: (i + _NBK, 0)),
                  pl.BlockSpec((_BK, 128), lambda i: (i, 0)),
                  pl.BlockSpec((_BK, 128), lambda i: (i + _NBK, 0)),
                  pl.BlockSpec((_BK, 128), lambda i: (i, 0)),
                  pl.BlockSpec((_BK, 128), lambda i: (i + _NBK, 0)),
                  pl.BlockSpec((128, 128), cmap),
                  pl.BlockSpec((128, 128), cmap),
                  pl.BlockSpec((1, 128), cmap)],
        out_specs=pl.BlockSpec((_BK, 128), lambda i: (i, 0)),
        out_shape=jax.ShapeDtypeStruct((KP, 128), F32),
    )(sums0, sums0, sums1, sums1, counts, counts, Wf0, Wf1, bf)


def kernel(coors, points, features, f_cluster, W_r0, g_r0, b_r0, W_r1, g_r1,
           b_r1, W_v0, g_v0, b_v0, W_v1, g_v1, b_v1, W_f, b_f):
    coors_i32 = coors.astype(jnp.int32)
    fc = f_cluster
    feat0 = jnp.concatenate([points, features], axis=1)
    r2 = lambda v: v.reshape(1, -1)

    inv, S1, M1 = _compute_inv(coors_i32, fc)
    S2, M2 = _run_kB(fc, S1, M1, W_r0, r2(g_r0), r2(b_r0))
    S3, M3 = _run_kC(fc, feat0, S1, M1, W_r0, r2(g_r0), r2(b_r0),
                     S2, M2, W_r1, r2(g_r1), r2(b_r1))
    pf0, Spf, Mpf = _run_kD(fc, feat0, S1, M1, W_r0, r2(g_r0), r2(b_r0),
                            S2, M2, W_r1, r2(g_r1), r2(b_r1),
                            S3, M3, W_v0, r2(g_v0), r2(b_v0))

    sums0 = _seg_sums_sc(pf0, inv)
    counts = _seg_counts_sc(inv)

    Wa = W_v1[:128]
    Wb = W_v1[128:]
    u, C = _run_kM(sums0, counts, Wb)

    g = _gather_sc(u, inv)
    pf1 = _run_kE(pf0, g, Wa, Wb, C, Mpf, Spf, r2(g_v1), r2(b_v1))

    sums1 = _seg_sums_sc(pf1, inv)

    vox = _run_kF(sums0, sums1, counts, W_f[:128], W_f[128:], r2(b_f))
    return pf1, vox[:K]
